# Initial kernel scaffold; baseline (speedup 1.0000x reference)
import jax
import jax.numpy as jnp
from jax import lax
from jax.experimental import pallas as pl
from jax.experimental.pallas import tpu as pltpu
from jax.experimental.pallas import tpu_sc as plsc

N = 10000
E = 160000
C = 256
NP = 10240
NT = 32
NPT = NP // NT
L = 16

CHUNK = 2000
NCHUNK = E // CHUNK
FLUSH = 2048
CB = FLUSH + 16
CAP = 160320

MCH = 480
GB = 48
NB = MCH // GB


def _mesh():
    return plsc.VectorSubcoreMesh(core_axis_name="c", subcore_axis_name="s")


def _wid():
    return lax.axis_index("s") * 2 + lax.axis_index("c")


def _partition_body(ei_hbm, ew_hbm, rows_hbm, cols_hbm, ews_hbm, cnt_hbm,
                    deg_hbm, row_in, col_in, ew_in, row_cb, col_cb, ew_cb,
                    deg_ln, deg_out, cnt_v, fill_sm, off_sm, sem0, sem1):
    wid = _wid()
    base = wid * NPT
    lanes = lax.iota(jnp.int32, L)

    zf = jnp.zeros((L,), jnp.float32)

    def zbody(j, _):
        for l in range(L):
            deg_ln[l, pl.ds(j * L, L)] = zf
        return 0
    lax.fori_loop(0, NPT // L, zbody, 0)
    fill_sm[0] = 0
    off_sm[0] = 0

    def issue(c, s):
        o = c * CHUNK
        sem = sem0 if s == 0 else sem1
        pltpu.async_copy(ei_hbm.at[0, pl.ds(o, CHUNK)], row_in.at[s], sem)
        pltpu.async_copy(ei_hbm.at[1, pl.ds(o, CHUNK)], col_in.at[s], sem)
        pltpu.async_copy(ew_hbm.at[pl.ds(o, CHUNK)], ew_in.at[s], sem)

    def wait(c, s):
        o = c * CHUNK
        sem = sem0 if s == 0 else sem1
        pltpu.make_async_copy(ei_hbm.at[0, pl.ds(o, CHUNK)],
                              row_in.at[s], sem).wait()
        pltpu.make_async_copy(ei_hbm.at[1, pl.ds(o, CHUNK)],
                              col_in.at[s], sem).wait()
        pltpu.make_async_copy(ew_hbm.at[pl.ds(o, CHUNK)],
                              ew_in.at[s], sem).wait()

    issue(0, 0)

    def chunk_body(c, _):
        slot = lax.rem(c, 2)

        @pl.when(c + 1 < NCHUNK)
        def _():
            lax.cond(slot == 0, lambda: issue(c + 1, 1),
                     lambda: issue(c + 1, 0))

        lax.cond(slot == 0, lambda: wait(c, 0), lambda: wait(c, 1))

        def pr(s):
            def gbody(g, _):
                col16 = col_in[s, pl.ds(g * L, L)]
                row16 = row_in[s, pl.ds(g * L, L)]
                ew16 = ew_in[s, pl.ds(g * L, L)]
                m = (col16 >= base) & (col16 < base + NPT)
                cl = jnp.where(m, col16 - base, 0)
                plsc.addupdate_scatter(deg_ln, [lanes, cl], ew16, mask=m)
                fill = fill_sm[0]
                plsc.store_compressed(row_cb.at[pl.ds(fill, L)], row16, mask=m)
                plsc.store_compressed(col_cb.at[pl.ds(fill, L)], col16, mask=m)
                plsc.store_compressed(ew_cb.at[pl.ds(fill, L)], ew16, mask=m)
                cnt = jnp.sum(m.astype(jnp.int32))
                fill = fill + cnt
                fill_sm[0] = fill

                @pl.when(fill >= FLUSH)
                def _():
                    oo = off_sm[0]
                    pltpu.sync_copy(row_cb.at[pl.ds(0, FLUSH)],
                                    rows_hbm.at[wid, pl.ds(oo, FLUSH)])
                    pltpu.sync_copy(col_cb.at[pl.ds(0, FLUSH)],
                                    cols_hbm.at[wid, pl.ds(oo, FLUSH)])
                    pltpu.sync_copy(ew_cb.at[pl.ds(0, FLUSH)],
                                    ews_hbm.at[wid, pl.ds(oo, FLUSH)])
                    r = fill - FLUSH

                    def mv(i, _):
                        row_cb[i] = row_cb[FLUSH + i]
                        col_cb[i] = col_cb[FLUSH + i]
                        ew_cb[i] = ew_cb[FLUSH + i]
                        return 0
                    lax.fori_loop(0, r, mv, 0)
                    fill_sm[0] = r
                    off_sm[0] = oo + FLUSH
                return 0

            lax.fori_loop(0, CHUNK // L, gbody, 0)

        lax.cond(slot == 0, lambda: pr(0), lambda: pr(1))
        return 0

    lax.fori_loop(0, NCHUNK, chunk_body, 0)

    fill = fill_sm[0]
    pad = lax.rem(64 - lax.rem(fill, 64), 64)

    def pbody(i, _):
        row_cb[fill + i] = 0
        col_cb[fill + i] = base
        ew_cb[fill + i] = 0.0
        return 0
    lax.fori_loop(0, pad, pbody, 0)
    fill = fill + pad

    def fbody(i, _):
        oo = off_sm[0]
        pltpu.sync_copy(row_cb.at[pl.ds(i * 64, 64)],
                        rows_hbm.at[wid, pl.ds(oo + i * 64, 64)])
        pltpu.sync_copy(col_cb.at[pl.ds(i * 64, 64)],
                        cols_hbm.at[wid, pl.ds(oo + i * 64, 64)])
        pltpu.sync_copy(ew_cb.at[pl.ds(i * 64, 64)],
                        ews_hbm.at[wid, pl.ds(oo + i * 64, 64)])
        return 0
    lax.fori_loop(0, fill // 64, fbody, 0)
    total = off_sm[0] + fill

    cnt_v[...] = jnp.broadcast_to(total, (L,)).astype(jnp.int32)
    pltpu.sync_copy(cnt_v, cnt_hbm.at[wid])

    for j in range(NPT // L):
        s = jnp.full((L,), 2.0, jnp.float32)
        for l in range(L):
            s = s + deg_ln[l, pl.ds(j * L, L)]
        deg_out[pl.ds(j * L, L)] = s
    pltpu.sync_copy(deg_out, deg_hbm.at[pl.ds(base, NPT)])


@jax.jit
def _sc_partition(edge_index, edge_weight):
    f = pl.kernel(
        _partition_body,
        out_type=[
            jax.ShapeDtypeStruct((NT, CAP), jnp.int32),
            jax.ShapeDtypeStruct((NT, CAP), jnp.int32),
            jax.ShapeDtypeStruct((NT, CAP), jnp.float32),
            jax.ShapeDtypeStruct((NT, L), jnp.int32),
            jax.ShapeDtypeStruct((NP,), jnp.float32),
        ],
        mesh=_mesh(),
        scratch_types=[
            pltpu.VMEM((2, CHUNK), jnp.int32),
            pltpu.VMEM((2, CHUNK), jnp.int32),
            pltpu.VMEM((2, CHUNK), jnp.float32),
            pltpu.VMEM((CB,), jnp.int32),
            pltpu.VMEM((CB,), jnp.int32),
            pltpu.VMEM((CB,), jnp.float32),
            pltpu.VMEM((L, NPT), jnp.float32),
            pltpu.VMEM((NPT,), jnp.float32),
            pltpu.VMEM((L,), jnp.int32),
            pltpu.SMEM((1,), jnp.int32),
            pltpu.SMEM((1,), jnp.int32),
            pltpu.SemaphoreType.DMA,
            pltpu.SemaphoreType.DMA,
        ],
    )
    return f(edge_index, edge_weight)


def _conv_body(y_hbm, rows_hbm, cols_hbm, ews_hbm, cnt_hbm, dis_hbm,
               msg_hbm, dis_v, cnt_v, row_m, col_m, ew_m, coeff_v, stage,
               acc, gsem0, gsem1):
    wid = _wid()
    base = wid * NPT
    lanes = lax.iota(jnp.int32, L)

    pltpu.sync_copy(dis_hbm, dis_v)
    pltpu.sync_copy(cnt_hbm.at[wid], cnt_v)
    total = cnt_v[0]

    zf = jnp.zeros((L,), jnp.float32)

    def zbody(j, _):
        for k in range(C // L):
            acc[j, pl.ds(k * L, L)] = zf
        return 0
    lax.fori_loop(0, NPT, zbody, 0)

    for b in range(10):
        pltpu.sync_copy(y_hbm.at[pl.ds(base + b * 32, 32)],
                        stage.at[0, pl.ds(0, 32)])

        def sbody(e, _):
            n = b * 32 + e
            d = dis_v[base + n]
            cvec = jnp.broadcast_to(2.0 * d * d, (L,))
            for j in range(C // L):
                plsc.addupdate(acc.at[n, pl.ds(j * L, L)],
                               cvec * stage[0, e, pl.ds(j * L, L)])
            return 0
        lax.fori_loop(0, 32, sbody, 0)

    nchunks = lax.div(total + (MCH - 1), MCH)

    def chunk_body(ci, _):
        co = ci * MCH
        pltpu.sync_copy(rows_hbm.at[wid, pl.ds(co, MCH)], row_m)
        pltpu.sync_copy(cols_hbm.at[wid, pl.ds(co, MCH)], col_m)
        pltpu.sync_copy(ews_hbm.at[wid, pl.ds(co, MCH)], ew_m)

        def coefb(g, _):
            gi = co + g * L + lanes
            m = gi < total
            r16 = jnp.where(m, row_m[pl.ds(g * L, L)], 0)
            c16 = jnp.where(m, col_m[pl.ds(g * L, L)], base)
            w16 = jnp.where(m, ew_m[pl.ds(g * L, L)], 0.0)
            dr = plsc.load_gather(dis_v, [r16])
            dc = plsc.load_gather(dis_v, [c16])
            row_m[pl.ds(g * L, L)] = r16
            col_m[pl.ds(g * L, L)] = c16
            coeff_v[pl.ds(g * L, L)] = w16 * dr * dc
            return 0
        lax.fori_loop(0, MCH // L, coefb, 0)

        def g_issue(b, s):
            sem = gsem0 if s == 0 else gsem1
            pltpu.async_copy(y_hbm.at[row_m.at[pl.ds(b * GB, GB)]],
                             stage.at[s], sem)

        def g_wait(b, s):
            sem = gsem0 if s == 0 else gsem1
            pltpu.make_async_copy(y_hbm.at[row_m.at[pl.ds(b * GB, GB)]],
                                  stage.at[s], sem).wait()

        g_issue(0, 0)
        for b in range(NB):
            s = b % 2
            if b + 1 < NB:
                g_issue(b + 1, 1 - s)
            g_wait(b, s)

            def ebody(e, _):
                i = b * GB + e
                cs = coeff_v[i]
                cl = col_m[i] - base
                cvec = jnp.broadcast_to(cs, (L,))
                for j in range(C // L):
                    plsc.addupdate(acc.at[cl, pl.ds(j * L, L)],
                                   cvec * stage[s, e, pl.ds(j * L, L)])
                return 0
            lax.fori_loop(0, GB, ebody, 0)
        return 0

    lax.fori_loop(0, nchunks, chunk_body, 0)

    pltpu.sync_copy(acc, msg_hbm.at[pl.ds(base, NPT)])


@jax.jit
def _sc_conv(y, rows_s, cols_s, ews_s, counts, dis):
    f = pl.kernel(
        _conv_body,
        out_type=[jax.ShapeDtypeStruct((NP, C), jnp.float32)],
        mesh=_mesh(),
        scratch_types=[
            pltpu.VMEM((NP,), jnp.float32),
            pltpu.VMEM((L,), jnp.int32),
            pltpu.VMEM((MCH,), jnp.int32),
            pltpu.VMEM((MCH,), jnp.int32),
            pltpu.VMEM((MCH,), jnp.float32),
            pltpu.VMEM((MCH,), jnp.float32),
            pltpu.VMEM((2, GB, C), jnp.float32),
            pltpu.VMEM((NPT, C), jnp.float32),
            pltpu.SemaphoreType.DMA,
            pltpu.SemaphoreType.DMA,
        ],
    )
    (msg,) = f(y, rows_s, cols_s, ews_s, counts, dis)
    return msg


def _tca_body(x_ref, w_ref, deg_ref, t_ref, wt_ref, bt_ref,
              y_ref, dis_ref, temb_ref):
    y_ref[...] = jnp.dot(x_ref[...], w_ref[...],
                         preferred_element_type=jnp.float32)
    dis_ref[...] = lax.rsqrt(deg_ref[...])
    temb_ref[...] = jax.nn.relu(
        jnp.dot(t_ref[...], wt_ref[...], preferred_element_type=jnp.float32)
        + bt_ref[...])


@jax.jit
def _tc_a(xp, W1, deg2d, t2, Wt, bt2):
    return pl.pallas_call(
        _tca_body,
        grid=(NP // 1024,),
        in_specs=[
            pl.BlockSpec((1024, C), lambda i: (i, 0)),
            pl.BlockSpec((C, C), lambda i: (0, 0)),
            pl.BlockSpec((8, 128), lambda i: (i, 0)),
            pl.BlockSpec((1, C), lambda i: (0, 0)),
            pl.BlockSpec((C, C), lambda i: (0, 0)),
            pl.BlockSpec((1, C), lambda i: (0, 0)),
        ],
        out_specs=[
            pl.BlockSpec((1024, C), lambda i: (i, 0)),
            pl.BlockSpec((8, 128), lambda i: (i, 0)),
            pl.BlockSpec((1, C), lambda i: (0, 0)),
        ],
        out_shape=[
            jax.ShapeDtypeStruct((NP, C), jnp.float32),
            jax.ShapeDtypeStruct((NP // 128, 128), jnp.float32),
            jax.ShapeDtypeStruct((1, C), jnp.float32),
        ],
    )(xp, W1, deg2d, t2, Wt, bt2)


def _ln(z, g, b):
    mu = jnp.mean(z, axis=-1, keepdims=True)
    var = jnp.mean((z - mu) ** 2, axis=-1, keepdims=True)
    return (z - mu) * lax.rsqrt(var + 1e-5) * g + b


def _tcb_body(msg_ref, b1_ref, g1_ref, be1_ref, temb_ref, w2_ref, y2_ref):
    z = jax.nn.relu(msg_ref[...] + b1_ref[...])
    h = _ln(z, g1_ref[...], be1_ref[...]) + temb_ref[...]
    y2_ref[...] = jnp.dot(h, w2_ref[...], preferred_element_type=jnp.float32)


@jax.jit
def _tc_b(msg1, b1r, g1r, be1r, temb, W2):
    return pl.pallas_call(
        _tcb_body,
        grid=(NP // 1024,),
        in_specs=[
            pl.BlockSpec((1024, C), lambda i: (i, 0)),
            pl.BlockSpec((1, C), lambda i: (0, 0)),
            pl.BlockSpec((1, C), lambda i: (0, 0)),
            pl.BlockSpec((1, C), lambda i: (0, 0)),
            pl.BlockSpec((1, C), lambda i: (0, 0)),
            pl.BlockSpec((C, C), lambda i: (0, 0)),
        ],
        out_specs=pl.BlockSpec((1024, C), lambda i: (i, 0)),
        out_shape=jax.ShapeDtypeStruct((NP, C), jnp.float32),
    )(msg1, b1r, g1r, be1r, temb, W2)


def _tcc_body(msg_ref, b2_ref, g2_ref, be2_ref, out_ref):
    z = jax.nn.relu(msg_ref[...] + b2_ref[...])
    out_ref[...] = _ln(z, g2_ref[...], be2_ref[...])


@jax.jit
def _tc_c(msg2, b2r, g2r, be2r):
    return pl.pallas_call(
        _tcc_body,
        grid=(NP // 1024,),
        in_specs=[
            pl.BlockSpec((1024, C), lambda i: (i, 0)),
            pl.BlockSpec((1, C), lambda i: (0, 0)),
            pl.BlockSpec((1, C), lambda i: (0, 0)),
            pl.BlockSpec((1, C), lambda i: (0, 0)),
        ],
        out_specs=pl.BlockSpec((1024, C), lambda i: (i, 0)),
        out_shape=jax.ShapeDtypeStruct((NP, C), jnp.float32),
    )(msg2, b2r, g2r, be2r)


def kernel(x, edge_index, edge_weight, t, W1, b1, g1, be1, W2, b2, g2, be2,
           Wt, bt):
    xp = jnp.pad(x, ((0, NP - N), (0, 0)))
    rows_s, cols_s, ews_s, counts, deg = _sc_partition(edge_index, edge_weight)
    y1, dis2d, temb = _tc_a(xp, W1, deg.reshape(NP // 128, 128),
                            t.reshape(1, C), Wt, bt.reshape(1, C))
    dis = dis2d.reshape(NP)
    msg1 = _sc_conv(y1, rows_s, cols_s, ews_s, counts, dis)
    y2 = _tc_b(msg1, b1.reshape(1, C), g1.reshape(1, C), be1.reshape(1, C),
               temb, W2)
    msg2 = _sc_conv(y2, rows_s, cols_s, ews_s, counts, dis)
    out = _tc_c(msg2, b2.reshape(1, C), g2.reshape(1, C), be2.reshape(1, C))
    return out[:N]


# speedup vs baseline: 3.3453x; 3.3453x over previous
import jax
import jax.numpy as jnp
from jax import lax
from jax.experimental import pallas as pl
from jax.experimental.pallas import tpu as pltpu
from jax.experimental.pallas import tpu_sc as plsc

N = 10000
E = 160000
C = 256
NP = 10240
NT = 32
NPT = NP // NT
L = 16

CHUNK = 2000
NCHUNK = E // CHUNK
FLUSH = 2048
CB = FLUSH + 80
CAP = 160320

MCH = 480
GB = 48
NB = MCH // GB


def _mesh():
    return plsc.VectorSubcoreMesh(core_axis_name="c", subcore_axis_name="s")


def _wid():
    return lax.axis_index("s") * 2 + lax.axis_index("c")


def _partition_body(rowe_hbm, cole_hbm, ew_hbm, rows_hbm, cols_hbm,
                    ews_hbm, cnt_hbm, deg_hbm, row_in0, row_in1,
                    col_in0, col_in1, ew_in0, ew_in1,
                    row_cb, col_cb, ew_cb, deg_ln, deg_out, cnt_v,
                    fill_sm, off_sm, sem0, sem1):
    wid = _wid()
    base = wid * NPT
    lanes = lax.iota(jnp.int32, L)

    zf = jnp.zeros((L,), jnp.float32)

    def zbody(j, _):
        deg_ln[pl.ds(j * L, L)] = zf
        return 0
    lax.fori_loop(0, (L * NPT) // L, zbody, 0)
    fill_sm[0] = 0
    off_sm[0] = 0

    bufs = ((row_in0, col_in0, ew_in0, sem0), (row_in1, col_in1, ew_in1, sem1))

    def issue(c, s):
        o = c * CHUNK
        ri, ci, wi, sem = bufs[s]
        pltpu.async_copy(rowe_hbm.at[pl.ds(o, CHUNK)], ri, sem)
        pltpu.async_copy(cole_hbm.at[pl.ds(o, CHUNK)], ci, sem)
        pltpu.async_copy(ew_hbm.at[pl.ds(o, CHUNK)], wi, sem)

    def wait(c, s):
        o = c * CHUNK
        ri, ci, wi, sem = bufs[s]
        pltpu.make_async_copy(rowe_hbm.at[pl.ds(o, CHUNK)], ri, sem).wait()
        pltpu.make_async_copy(cole_hbm.at[pl.ds(o, CHUNK)], ci, sem).wait()
        pltpu.make_async_copy(ew_hbm.at[pl.ds(o, CHUNK)], wi, sem).wait()

    issue(0, 0)

    def chunk_body(c, _):
        slot = lax.rem(c, 2)

        @pl.when(c + 1 < NCHUNK)
        def _():
            lax.cond(slot == 0, lambda: issue(c + 1, 1),
                     lambda: issue(c + 1, 0))

        lax.cond(slot == 0, lambda: wait(c, 0), lambda: wait(c, 1))

        def pr(s):
            ri, ci, wi, _sem = bufs[s]

            def gbody(g, _):
                col16 = ci[pl.ds(g * L, L)]
                row16 = ri[pl.ds(g * L, L)]
                ew16 = wi[pl.ds(g * L, L)]
                m = (col16 >= base) & (col16 < base + NPT)
                cl = jnp.where(m, col16 - base, 0)
                plsc.addupdate_scatter(deg_ln, [lanes * NPT + cl], ew16, mask=m)
                fill = fill_sm[0]
                plsc.store_compressed(row_cb.at[pl.ds(fill, L)], row16, mask=m)
                plsc.store_compressed(col_cb.at[pl.ds(fill, L)], col16, mask=m)
                plsc.store_compressed(ew_cb.at[pl.ds(fill, L)], ew16, mask=m)
                cnt = jnp.sum(m.astype(jnp.int32))
                fill = fill + cnt
                fill_sm[0] = fill

                @pl.when(fill >= FLUSH)
                def _():
                    oo = off_sm[0]
                    pltpu.sync_copy(row_cb.at[pl.ds(0, FLUSH)],
                                    rows_hbm.at[pl.ds(pl.multiple_of(wid * CAP + oo, 64), FLUSH)])
                    pltpu.sync_copy(col_cb.at[pl.ds(0, FLUSH)],
                                    cols_hbm.at[pl.ds(pl.multiple_of(wid * CAP + oo, 64), FLUSH)])
                    pltpu.sync_copy(ew_cb.at[pl.ds(0, FLUSH)],
                                    ews_hbm.at[pl.ds(pl.multiple_of(wid * CAP + oo, 64), FLUSH)])
                    row_cb[pl.ds(0, L)] = row_cb[pl.ds(FLUSH, L)]
                    col_cb[pl.ds(0, L)] = col_cb[pl.ds(FLUSH, L)]
                    ew_cb[pl.ds(0, L)] = ew_cb[pl.ds(FLUSH, L)]
                    fill_sm[0] = fill - FLUSH
                    off_sm[0] = oo + FLUSH
                return 0

            lax.fori_loop(0, CHUNK // L, gbody, 0)

        lax.cond(slot == 0, lambda: pr(0), lambda: pr(1))
        return 0

    lax.fori_loop(0, NCHUNK, chunk_body, 0)

    fill = fill_sm[0]
    pad = lax.rem(64 - lax.rem(fill, 64), 64)
    zi = jnp.zeros((L,), jnp.int32)
    bv = jnp.full((L,), 1, jnp.int32) * base
    for k in range(4):
        row_cb[pl.ds(fill + k * L, L)] = zi
        col_cb[pl.ds(fill + k * L, L)] = bv
        ew_cb[pl.ds(fill + k * L, L)] = zf
    fill = fill + pad

    def fbody(i, _):
        oo = off_sm[0]
        pltpu.sync_copy(row_cb.at[pl.ds(i * 64, 64)],
                        rows_hbm.at[pl.ds(pl.multiple_of(wid * CAP + oo + i * 64, 64), 64)])
        pltpu.sync_copy(col_cb.at[pl.ds(i * 64, 64)],
                        cols_hbm.at[pl.ds(pl.multiple_of(wid * CAP + oo + i * 64, 64), 64)])
        pltpu.sync_copy(ew_cb.at[pl.ds(i * 64, 64)],
                        ews_hbm.at[pl.ds(pl.multiple_of(wid * CAP + oo + i * 64, 64), 64)])
        return 0
    lax.fori_loop(0, fill // 64, fbody, 0)
    total = off_sm[0] + fill

    cnt_v[...] = jnp.broadcast_to(total, (L,)).astype(jnp.int32)
    pltpu.sync_copy(cnt_v, cnt_hbm.at[pl.ds(pl.multiple_of(wid * L, L), L)])

    for j in range(NPT // L):
        s = jnp.full((L,), 2.0, jnp.float32)
        for l in range(L):
            s = s + deg_ln[pl.ds(l * NPT + j * L, L)]
        deg_out[pl.ds(j * L, L)] = s
    pltpu.sync_copy(deg_out, deg_hbm.at[pl.ds(pl.multiple_of(base, 64), NPT)])


@jax.jit
def _sc_partition(row_e, col_e, edge_weight):
    f = pl.kernel(
        _partition_body,
        out_type=[
            jax.ShapeDtypeStruct((NT * CAP,), jnp.int32),
            jax.ShapeDtypeStruct((NT * CAP,), jnp.int32),
            jax.ShapeDtypeStruct((NT * CAP,), jnp.float32),
            jax.ShapeDtypeStruct((NT * L,), jnp.int32),
            jax.ShapeDtypeStruct((NP,), jnp.float32),
        ],
        mesh=_mesh(),
        compiler_params=pltpu.CompilerParams(needs_layout_passes=False),
        scratch_types=[
            pltpu.VMEM((CHUNK,), jnp.int32),
            pltpu.VMEM((CHUNK,), jnp.int32),
            pltpu.VMEM((CHUNK,), jnp.int32),
            pltpu.VMEM((CHUNK,), jnp.int32),
            pltpu.VMEM((CHUNK,), jnp.float32),
            pltpu.VMEM((CHUNK,), jnp.float32),
            pltpu.VMEM((CB,), jnp.int32),
            pltpu.VMEM((CB,), jnp.int32),
            pltpu.VMEM((CB,), jnp.float32),
            pltpu.VMEM((L * NPT,), jnp.float32),
            pltpu.VMEM((NPT,), jnp.float32),
            pltpu.VMEM((L,), jnp.int32),
            pltpu.SMEM((1,), jnp.int32),
            pltpu.SMEM((1,), jnp.int32),
            pltpu.SemaphoreType.DMA,
            pltpu.SemaphoreType.DMA,
        ],
    )
    return f(row_e, col_e, edge_weight)


def _conv_body(y_hbm, rows_hbm, cols_hbm, ews_hbm, cnt_hbm, dis_hbm,
               msg_hbm, dis_v, cnt_v, row_m, col_m, ew_m, coeff_v,
               stage0, stage1, acc, gsem0, gsem1):
    wid = _wid()
    base = wid * NPT
    lanes = lax.iota(jnp.int32, L)

    pltpu.sync_copy(dis_hbm, dis_v)
    pltpu.sync_copy(cnt_hbm.at[pl.ds(pl.multiple_of(wid * L, L), L)], cnt_v)
    total = cnt_v[...][0]

    zf = jnp.zeros((L,), jnp.float32)

    def zbody(j, _):
        for k in range(C // L):
            acc[j, pl.ds(k * L, L)] = zf
        return 0
    lax.fori_loop(0, NPT, zbody, 0)

    def selfb(bb, _):
        pltpu.sync_copy(y_hbm.at[pl.ds(pl.multiple_of(base + bb * 32, 32), 32)],
                        stage0.at[pl.ds(0, 32)])

        def sbody(g, _):
            d16 = dis_v[pl.ds(base + bb * 32 + g * L, L)]
            c16 = 2.0 * d16 * d16
            for e in range(L):
                n = g * L + e
                cvec = jnp.broadcast_to(c16[e], (L,))
                for j in range(C // L):
                    plsc.addupdate(acc.at[bb * 32 + n, pl.ds(j * L, L)],
                                   cvec * stage0[n, pl.ds(j * L, L)])
            return 0
        lax.fori_loop(0, 2, sbody, 0)
        return 0
    lax.fori_loop(0, NPT // 32, selfb, 0)

    nchunks = lax.div(total + (MCH - 1), MCH)

    def chunk_body(ci, _):
        co = ci * MCH
        pltpu.sync_copy(rows_hbm.at[pl.ds(pl.multiple_of(wid * CAP + co, 16), MCH)], row_m)
        pltpu.sync_copy(cols_hbm.at[pl.ds(pl.multiple_of(wid * CAP + co, 16), MCH)], col_m)
        pltpu.sync_copy(ews_hbm.at[pl.ds(pl.multiple_of(wid * CAP + co, 16), MCH)], ew_m)

        def coefb(g, _):
            gi = co + g * L + lanes
            m = gi < total
            r16 = jnp.where(m, row_m[pl.ds(g * L, L)], 0)
            c16 = jnp.where(m, col_m[pl.ds(g * L, L)], base)
            w16 = jnp.where(m, ew_m[pl.ds(g * L, L)], 0.0)
            dr = plsc.load_gather(dis_v, [r16])
            dc = plsc.load_gather(dis_v, [c16])
            row_m[pl.ds(g * L, L)] = r16
            col_m[pl.ds(g * L, L)] = c16
            coeff_v[pl.ds(g * L, L)] = w16 * dr * dc
            return 0
        lax.fori_loop(0, MCH // L, coefb, 0)

        gbufs = ((stage0, gsem0), (stage1, gsem1))

        def g_issue(b, s):
            st, sem = gbufs[s]
            pltpu.async_copy(y_hbm.at[row_m.at[pl.ds(b * GB, GB)]], st, sem)

        def g_wait(b, s):
            st, sem = gbufs[s]
            pltpu.make_async_copy(y_hbm.at[row_m.at[pl.ds(b * GB, GB)]],
                                  st, sem).wait()

        g_issue(0, 0)

        def batch_body(b, _):
            slot = lax.rem(b, 2)

            def proc(s):
                g_wait(b, s)

                @pl.when(b + 1 < NB)
                def _():
                    g_issue(b + 1, 1 - s)

                def ebody(g, _):
                    i0 = b * GB + g * L
                    c16 = coeff_v[pl.ds(i0, L)]
                    cl16 = col_m[pl.ds(i0, L)] - base
                    for e in range(L):
                        cvec = jnp.broadcast_to(c16[e], (L,))
                        cl = cl16[e]
                        for j in range(C // L):
                            plsc.addupdate(
                                acc.at[cl, pl.ds(j * L, L)],
                                cvec * gbufs[s][0][g * L + e, pl.ds(j * L, L)])
                    return 0
                lax.fori_loop(0, GB // L, ebody, 0)

            lax.cond(slot == 0, lambda: proc(0), lambda: proc(1))
            return 0
        lax.fori_loop(0, NB, batch_body, 0)
        return 0

    lax.fori_loop(0, nchunks, chunk_body, 0)

    pltpu.sync_copy(acc, msg_hbm.at[pl.ds(pl.multiple_of(base, 64), NPT)])


@jax.jit
def _sc_conv(y, rows_s, cols_s, ews_s, counts, dis):
    f = pl.kernel(
        _conv_body,
        out_type=[jax.ShapeDtypeStruct((NP, C), jnp.float32)],
        mesh=_mesh(),
        compiler_params=pltpu.CompilerParams(needs_layout_passes=False),
        scratch_types=[
            pltpu.VMEM((NP,), jnp.float32),
            pltpu.VMEM((L,), jnp.int32),
            pltpu.VMEM((MCH,), jnp.int32),
            pltpu.VMEM((MCH,), jnp.int32),
            pltpu.VMEM((MCH,), jnp.float32),
            pltpu.VMEM((MCH,), jnp.float32),
            pltpu.VMEM((GB, C), jnp.float32),
            pltpu.VMEM((GB, C), jnp.float32),
            pltpu.VMEM((NPT, C), jnp.float32),
            pltpu.SemaphoreType.DMA,
            pltpu.SemaphoreType.DMA,
        ],
    )
    (msg,) = f(y, rows_s, cols_s, ews_s, counts, dis)
    return msg


def _tca_body(x_ref, w_ref, deg_ref, t_ref, wt_ref, bt_ref,
              y_ref, dis_ref, temb_ref):
    y_ref[...] = jnp.dot(x_ref[...], w_ref[...],
                         preferred_element_type=jnp.float32)
    dis_ref[...] = lax.rsqrt(deg_ref[...])
    temb_ref[...] = jax.nn.relu(
        jnp.dot(t_ref[...], wt_ref[...], preferred_element_type=jnp.float32)
        + bt_ref[...])


@jax.jit
def _tc_a(xp, W1, deg2d, t2, Wt, bt2):
    return pl.pallas_call(
        _tca_body,
        grid=(NP // 1024,),
        in_specs=[
            pl.BlockSpec((1024, C), lambda i: (i, 0)),
            pl.BlockSpec((C, C), lambda i: (0, 0)),
            pl.BlockSpec((8, 128), lambda i: (i, 0)),
            pl.BlockSpec((1, C), lambda i: (0, 0)),
            pl.BlockSpec((C, C), lambda i: (0, 0)),
            pl.BlockSpec((1, C), lambda i: (0, 0)),
        ],
        out_specs=[
            pl.BlockSpec((1024, C), lambda i: (i, 0)),
            pl.BlockSpec((8, 128), lambda i: (i, 0)),
            pl.BlockSpec((1, C), lambda i: (0, 0)),
        ],
        out_shape=[
            jax.ShapeDtypeStruct((NP, C), jnp.float32),
            jax.ShapeDtypeStruct((NP // 128, 128), jnp.float32),
            jax.ShapeDtypeStruct((1, C), jnp.float32),
        ],
    )(xp, W1, deg2d, t2, Wt, bt2)


def _ln(z, g, b):
    mu = jnp.mean(z, axis=-1, keepdims=True)
    var = jnp.mean((z - mu) ** 2, axis=-1, keepdims=True)
    return (z - mu) * lax.rsqrt(var + 1e-5) * g + b


def _tcb_body(msg_ref, b1_ref, g1_ref, be1_ref, temb_ref, w2_ref, y2_ref):
    z = jax.nn.relu(msg_ref[...] + b1_ref[...])
    h = _ln(z, g1_ref[...], be1_ref[...]) + temb_ref[...]
    y2_ref[...] = jnp.dot(h, w2_ref[...], preferred_element_type=jnp.float32)


@jax.jit
def _tc_b(msg1, b1r, g1r, be1r, temb, W2):
    return pl.pallas_call(
        _tcb_body,
        grid=(NP // 1024,),
        in_specs=[
            pl.BlockSpec((1024, C), lambda i: (i, 0)),
            pl.BlockSpec((1, C), lambda i: (0, 0)),
            pl.BlockSpec((1, C), lambda i: (0, 0)),
            pl.BlockSpec((1, C), lambda i: (0, 0)),
            pl.BlockSpec((1, C), lambda i: (0, 0)),
            pl.BlockSpec((C, C), lambda i: (0, 0)),
        ],
        out_specs=pl.BlockSpec((1024, C), lambda i: (i, 0)),
        out_shape=jax.ShapeDtypeStruct((NP, C), jnp.float32),
    )(msg1, b1r, g1r, be1r, temb, W2)


def _tcc_body(msg_ref, b2_ref, g2_ref, be2_ref, out_ref):
    z = jax.nn.relu(msg_ref[...] + b2_ref[...])
    out_ref[...] = _ln(z, g2_ref[...], be2_ref[...])


@jax.jit
def _tc_c(msg2, b2r, g2r, be2r):
    return pl.pallas_call(
        _tcc_body,
        grid=(NP // 1024,),
        in_specs=[
            pl.BlockSpec((1024, C), lambda i: (i, 0)),
            pl.BlockSpec((1, C), lambda i: (0, 0)),
            pl.BlockSpec((1, C), lambda i: (0, 0)),
            pl.BlockSpec((1, C), lambda i: (0, 0)),
        ],
        out_specs=pl.BlockSpec((1024, C), lambda i: (i, 0)),
        out_shape=jax.ShapeDtypeStruct((NP, C), jnp.float32),
    )(msg2, b2r, g2r, be2r)


def kernel(x, edge_index, edge_weight, t, W1, b1, g1, be1, W2, b2, g2, be2,
           Wt, bt):
    xp = jnp.pad(x, ((0, NP - N), (0, 0)))
    rows_s, cols_s, ews_s, counts, deg = _sc_partition(
        edge_index[0], edge_index[1], edge_weight)
    y1, dis2d, temb = _tc_a(xp, W1, deg.reshape(NP // 128, 128),
                            t.reshape(1, C), Wt, bt.reshape(1, C))
    dis = dis2d.reshape(NP)
    msg1 = _sc_conv(y1, rows_s, cols_s, ews_s, counts, dis)
    y2 = _tc_b(msg1, b1.reshape(1, C), g1.reshape(1, C), be1.reshape(1, C),
               temb, W2)
    msg2 = _sc_conv(y2, rows_s, cols_s, ews_s, counts, dis)
    out = _tc_c(msg2, b2.reshape(1, C), g2.reshape(1, C), be2.reshape(1, C))
    return out[:N]


# speedup vs baseline: 4.4714x; 1.3366x over previous
import jax
import jax.numpy as jnp
from jax import lax
from jax.experimental import pallas as pl
from jax.experimental.pallas import tpu as pltpu
from jax.experimental.pallas import tpu_sc as plsc

N = 10000
E = 160000
C = 256
NP = 10240
NT = 32
NPT = NP // NT
L = 16

CHUNK = 2000
NCHUNK = E // CHUNK
FLUSH = 2048
CB = FLUSH + 80
CAP = 160320

MCH = 480
GB = 48
NB = MCH // GB


def _mesh():
    return plsc.VectorSubcoreMesh(core_axis_name="c", subcore_axis_name="s")


def _wid():
    return lax.axis_index("s") * 2 + lax.axis_index("c")


def _partition_body(rowe_hbm, cole_hbm, ew_hbm, rows_hbm, cols_hbm,
                    ews_hbm, cnt_hbm, deg_hbm, row_in0, row_in1,
                    col_in0, col_in1, ew_in0, ew_in1,
                    row_cb, col_cb, ew_cb, deg_ln, deg_out, cnt_v,
                    fill_sm, off_sm, sem0, sem1):
    wid = _wid()
    base = wid * NPT
    lanes = lax.iota(jnp.int32, L)

    zf = jnp.zeros((L,), jnp.float32)

    def zbody(j, _):
        deg_ln[pl.ds(j * L, L)] = zf
        return 0
    lax.fori_loop(0, (L * NPT) // L, zbody, 0)
    fill_sm[0] = 0
    off_sm[0] = 0

    bufs = ((row_in0, col_in0, ew_in0, sem0), (row_in1, col_in1, ew_in1, sem1))

    def issue(c, s):
        o = c * CHUNK
        ri, ci, wi, sem = bufs[s]
        pltpu.async_copy(rowe_hbm.at[pl.ds(o, CHUNK)], ri, sem)
        pltpu.async_copy(cole_hbm.at[pl.ds(o, CHUNK)], ci, sem)
        pltpu.async_copy(ew_hbm.at[pl.ds(o, CHUNK)], wi, sem)

    def wait(c, s):
        o = c * CHUNK
        ri, ci, wi, sem = bufs[s]
        pltpu.make_async_copy(rowe_hbm.at[pl.ds(o, CHUNK)], ri, sem).wait()
        pltpu.make_async_copy(cole_hbm.at[pl.ds(o, CHUNK)], ci, sem).wait()
        pltpu.make_async_copy(ew_hbm.at[pl.ds(o, CHUNK)], wi, sem).wait()

    issue(0, 0)

    def chunk_body(c, _):
        slot = lax.rem(c, 2)

        @pl.when(c + 1 < NCHUNK)
        def _():
            lax.cond(slot == 0, lambda: issue(c + 1, 1),
                     lambda: issue(c + 1, 0))

        lax.cond(slot == 0, lambda: wait(c, 0), lambda: wait(c, 1))

        def pr(s):
            ri, ci, wi, _sem = bufs[s]

            def gbody(g, _):
                col16 = ci[pl.ds(g * L, L)]
                row16 = ri[pl.ds(g * L, L)]
                ew16 = wi[pl.ds(g * L, L)]
                m = (col16 >= base) & (col16 < base + NPT)
                cl = jnp.where(m, col16 - base, 0)
                plsc.addupdate_scatter(deg_ln, [lanes * NPT + cl], ew16, mask=m)
                fill = fill_sm[0]
                plsc.store_compressed(row_cb.at[pl.ds(fill, L)], row16, mask=m)
                plsc.store_compressed(col_cb.at[pl.ds(fill, L)], col16, mask=m)
                plsc.store_compressed(ew_cb.at[pl.ds(fill, L)], ew16, mask=m)
                cnt = jnp.sum(m.astype(jnp.int32))
                fill = fill + cnt
                fill_sm[0] = fill

                @pl.when(fill >= FLUSH)
                def _():
                    oo = off_sm[0]
                    pltpu.sync_copy(row_cb.at[pl.ds(0, FLUSH)],
                                    rows_hbm.at[pl.ds(pl.multiple_of(wid * CAP + oo, 64), FLUSH)])
                    pltpu.sync_copy(col_cb.at[pl.ds(0, FLUSH)],
                                    cols_hbm.at[pl.ds(pl.multiple_of(wid * CAP + oo, 64), FLUSH)])
                    pltpu.sync_copy(ew_cb.at[pl.ds(0, FLUSH)],
                                    ews_hbm.at[pl.ds(pl.multiple_of(wid * CAP + oo, 64), FLUSH)])
                    row_cb[pl.ds(0, L)] = row_cb[pl.ds(FLUSH, L)]
                    col_cb[pl.ds(0, L)] = col_cb[pl.ds(FLUSH, L)]
                    ew_cb[pl.ds(0, L)] = ew_cb[pl.ds(FLUSH, L)]
                    fill_sm[0] = fill - FLUSH
                    off_sm[0] = oo + FLUSH
                return 0

            lax.fori_loop(0, CHUNK // L, gbody, 0)

        lax.cond(slot == 0, lambda: pr(0), lambda: pr(1))
        return 0

    lax.fori_loop(0, NCHUNK, chunk_body, 0)

    fill = fill_sm[0]
    pad = lax.rem(64 - lax.rem(fill, 64), 64)
    zi = jnp.zeros((L,), jnp.int32)
    bv = jnp.full((L,), 1, jnp.int32) * base
    for k in range(4):
        row_cb[pl.ds(fill + k * L, L)] = zi
        col_cb[pl.ds(fill + k * L, L)] = bv
        ew_cb[pl.ds(fill + k * L, L)] = zf
    fill = fill + pad

    def fbody(i, _):
        oo = off_sm[0]
        pltpu.sync_copy(row_cb.at[pl.ds(i * 64, 64)],
                        rows_hbm.at[pl.ds(pl.multiple_of(wid * CAP + oo + i * 64, 64), 64)])
        pltpu.sync_copy(col_cb.at[pl.ds(i * 64, 64)],
                        cols_hbm.at[pl.ds(pl.multiple_of(wid * CAP + oo + i * 64, 64), 64)])
        pltpu.sync_copy(ew_cb.at[pl.ds(i * 64, 64)],
                        ews_hbm.at[pl.ds(pl.multiple_of(wid * CAP + oo + i * 64, 64), 64)])
        return 0
    lax.fori_loop(0, fill // 64, fbody, 0)
    total = off_sm[0] + fill

    cnt_v[...] = jnp.broadcast_to(total, (L,)).astype(jnp.int32)
    pltpu.sync_copy(cnt_v, cnt_hbm.at[pl.ds(pl.multiple_of(wid * L, L), L)])

    for j in range(NPT // L):
        s = jnp.full((L,), 2.0, jnp.float32)
        for l in range(L):
            s = s + deg_ln[pl.ds(l * NPT + j * L, L)]
        deg_out[pl.ds(j * L, L)] = s
    pltpu.sync_copy(deg_out, deg_hbm.at[pl.ds(pl.multiple_of(base, 64), NPT)])


@jax.jit
def _sc_partition(row_e, col_e, edge_weight):
    f = pl.kernel(
        _partition_body,
        out_type=[
            jax.ShapeDtypeStruct((NT * CAP,), jnp.int32),
            jax.ShapeDtypeStruct((NT * CAP,), jnp.int32),
            jax.ShapeDtypeStruct((NT * CAP,), jnp.float32),
            jax.ShapeDtypeStruct((NT * L,), jnp.int32),
            jax.ShapeDtypeStruct((NP,), jnp.float32),
        ],
        mesh=_mesh(),
        compiler_params=pltpu.CompilerParams(needs_layout_passes=False),
        scratch_types=[
            pltpu.VMEM((CHUNK,), jnp.int32),
            pltpu.VMEM((CHUNK,), jnp.int32),
            pltpu.VMEM((CHUNK,), jnp.int32),
            pltpu.VMEM((CHUNK,), jnp.int32),
            pltpu.VMEM((CHUNK,), jnp.float32),
            pltpu.VMEM((CHUNK,), jnp.float32),
            pltpu.VMEM((CB,), jnp.int32),
            pltpu.VMEM((CB,), jnp.int32),
            pltpu.VMEM((CB,), jnp.float32),
            pltpu.VMEM((L * NPT,), jnp.float32),
            pltpu.VMEM((NPT,), jnp.float32),
            pltpu.VMEM((L,), jnp.int32),
            pltpu.SMEM((1,), jnp.int32),
            pltpu.SMEM((1,), jnp.int32),
            pltpu.SemaphoreType.DMA,
            pltpu.SemaphoreType.DMA,
        ],
    )
    return f(row_e, col_e, edge_weight)


def _conv_body(y_hbm, rows_hbm, cols_hbm, ews_hbm, cnt_hbm, dis_hbm,
               msg_hbm, dis_v, cnt_v, row_m, col_m, ew_m, coeff_v,
               stage0, stage1, acc, gsem0, gsem1):
    wid = _wid()
    base = wid * NPT
    lanes = lax.iota(jnp.int32, L)

    pltpu.sync_copy(dis_hbm, dis_v)
    pltpu.sync_copy(cnt_hbm.at[pl.ds(pl.multiple_of(wid * L, L), L)], cnt_v)
    total = cnt_v[...][0]

    zf = jnp.zeros((L,), jnp.float32)

    def zbody(j, _):
        for k in range(C // L):
            acc[j, pl.ds(k * L, L)] = zf
        return 0
    lax.fori_loop(0, NPT, zbody, 0)

    def selfb(bb, _):
        pltpu.sync_copy(y_hbm.at[pl.ds(pl.multiple_of(base + bb * 32, 32), 32)],
                        stage0.at[pl.ds(0, 32)])

        def sbody(g, _):
            d16 = dis_v[pl.ds(base + bb * 32 + g * L, L)]
            c16 = 2.0 * d16 * d16
            for e in range(L):
                n = g * L + e
                cvec = jnp.broadcast_to(c16[e], (L,))
                vals = [stage0[n, pl.ds(j * L, L)] for j in range(C // L)]
                for j in range(C // L):
                    plsc.addupdate(acc.at[bb * 32 + n, pl.ds(j * L, L)],
                                   cvec * vals[j])
            return 0
        lax.fori_loop(0, 2, sbody, 0)
        return 0
    lax.fori_loop(0, NPT // 32, selfb, 0)

    nchunks = lax.div(total + (MCH - 1), MCH)

    def chunk_body(ci, _):
        co = ci * MCH
        pltpu.sync_copy(rows_hbm.at[pl.ds(pl.multiple_of(wid * CAP + co, 16), MCH)], row_m)
        pltpu.sync_copy(cols_hbm.at[pl.ds(pl.multiple_of(wid * CAP + co, 16), MCH)], col_m)
        pltpu.sync_copy(ews_hbm.at[pl.ds(pl.multiple_of(wid * CAP + co, 16), MCH)], ew_m)

        def coefb(g, _):
            gi = co + g * L + lanes
            m = gi < total
            r16 = jnp.where(m, row_m[pl.ds(g * L, L)], 0)
            c16 = jnp.where(m, col_m[pl.ds(g * L, L)], base)
            w16 = jnp.where(m, ew_m[pl.ds(g * L, L)], 0.0)
            dr = plsc.load_gather(dis_v, [r16])
            dc = plsc.load_gather(dis_v, [c16])
            row_m[pl.ds(g * L, L)] = r16
            col_m[pl.ds(g * L, L)] = c16
            coeff_v[pl.ds(g * L, L)] = w16 * dr * dc
            return 0
        lax.fori_loop(0, MCH // L, coefb, 0)

        gbufs = ((stage0, gsem0), (stage1, gsem1))

        def g_issue(b, s):
            st, sem = gbufs[s]
            pltpu.async_copy(y_hbm.at[row_m.at[pl.ds(b * GB, GB)]], st, sem)

        def g_wait(b, s):
            st, sem = gbufs[s]
            pltpu.make_async_copy(y_hbm.at[row_m.at[pl.ds(b * GB, GB)]],
                                  st, sem).wait()

        g_issue(0, 0)

        def batch_body(b, _):
            slot = lax.rem(b, 2)

            def proc(s):
                g_wait(b, s)

                @pl.when(b + 1 < NB)
                def _():
                    g_issue(b + 1, 1 - s)

                def ebody(g):
                    i0 = b * GB + g * L
                    c16 = coeff_v[pl.ds(i0, L)]
                    cl16 = col_m[pl.ds(i0, L)] - base
                    for e in range(L):
                        cvec = jnp.broadcast_to(c16[e], (L,))
                        cl = cl16[e]
                        vals = [gbufs[s][0][g * L + e, pl.ds(j * L, L)]
                                for j in range(C // L)]
                        for j in range(C // L):
                            plsc.addupdate(acc.at[cl, pl.ds(j * L, L)],
                                           cvec * vals[j])
                plsc.parallel_loop(0, GB // L)(ebody)

            lax.cond(slot == 0, lambda: proc(0), lambda: proc(1))
            return 0
        lax.fori_loop(0, NB, batch_body, 0)
        return 0

    lax.fori_loop(0, nchunks, chunk_body, 0)

    pltpu.sync_copy(acc, msg_hbm.at[pl.ds(pl.multiple_of(base, 64), NPT)])


@jax.jit
def _sc_conv(y, rows_s, cols_s, ews_s, counts, dis):
    f = pl.kernel(
        _conv_body,
        out_type=[jax.ShapeDtypeStruct((NP, C), jnp.float32)],
        mesh=_mesh(),
        compiler_params=pltpu.CompilerParams(needs_layout_passes=False),
        scratch_types=[
            pltpu.VMEM((NP,), jnp.float32),
            pltpu.VMEM((L,), jnp.int32),
            pltpu.VMEM((MCH,), jnp.int32),
            pltpu.VMEM((MCH,), jnp.int32),
            pltpu.VMEM((MCH,), jnp.float32),
            pltpu.VMEM((MCH,), jnp.float32),
            pltpu.VMEM((GB, C), jnp.float32),
            pltpu.VMEM((GB, C), jnp.float32),
            pltpu.VMEM((NPT, C), jnp.float32),
            pltpu.SemaphoreType.DMA,
            pltpu.SemaphoreType.DMA,
        ],
    )
    (msg,) = f(y, rows_s, cols_s, ews_s, counts, dis)
    return msg


def _tca_body(x_ref, w_ref, deg_ref, t_ref, wt_ref, bt_ref,
              y_ref, dis_ref, temb_ref):
    y_ref[...] = jnp.dot(x_ref[...], w_ref[...],
                         preferred_element_type=jnp.float32)
    dis_ref[...] = lax.rsqrt(deg_ref[...])
    temb_ref[...] = jax.nn.relu(
        jnp.dot(t_ref[...], wt_ref[...], preferred_element_type=jnp.float32)
        + bt_ref[...])


@jax.jit
def _tc_a(xp, W1, deg2d, t2, Wt, bt2):
    return pl.pallas_call(
        _tca_body,
        grid=(NP // 1024,),
        in_specs=[
            pl.BlockSpec((1024, C), lambda i: (i, 0)),
            pl.BlockSpec((C, C), lambda i: (0, 0)),
            pl.BlockSpec((8, 128), lambda i: (i, 0)),
            pl.BlockSpec((1, C), lambda i: (0, 0)),
            pl.BlockSpec((C, C), lambda i: (0, 0)),
            pl.BlockSpec((1, C), lambda i: (0, 0)),
        ],
        out_specs=[
            pl.BlockSpec((1024, C), lambda i: (i, 0)),
            pl.BlockSpec((8, 128), lambda i: (i, 0)),
            pl.BlockSpec((1, C), lambda i: (0, 0)),
        ],
        out_shape=[
            jax.ShapeDtypeStruct((NP, C), jnp.float32),
            jax.ShapeDtypeStruct((NP // 128, 128), jnp.float32),
            jax.ShapeDtypeStruct((1, C), jnp.float32),
        ],
    )(xp, W1, deg2d, t2, Wt, bt2)


def _ln(z, g, b):
    mu = jnp.mean(z, axis=-1, keepdims=True)
    var = jnp.mean((z - mu) ** 2, axis=-1, keepdims=True)
    return (z - mu) * lax.rsqrt(var + 1e-5) * g + b


def _tcb_body(msg_ref, b1_ref, g1_ref, be1_ref, temb_ref, w2_ref, y2_ref):
    z = jax.nn.relu(msg_ref[...] + b1_ref[...])
    h = _ln(z, g1_ref[...], be1_ref[...]) + temb_ref[...]
    y2_ref[...] = jnp.dot(h, w2_ref[...], preferred_element_type=jnp.float32)


@jax.jit
def _tc_b(msg1, b1r, g1r, be1r, temb, W2):
    return pl.pallas_call(
        _tcb_body,
        grid=(NP // 1024,),
        in_specs=[
            pl.BlockSpec((1024, C), lambda i: (i, 0)),
            pl.BlockSpec((1, C), lambda i: (0, 0)),
            pl.BlockSpec((1, C), lambda i: (0, 0)),
            pl.BlockSpec((1, C), lambda i: (0, 0)),
            pl.BlockSpec((1, C), lambda i: (0, 0)),
            pl.BlockSpec((C, C), lambda i: (0, 0)),
        ],
        out_specs=pl.BlockSpec((1024, C), lambda i: (i, 0)),
        out_shape=jax.ShapeDtypeStruct((NP, C), jnp.float32),
    )(msg1, b1r, g1r, be1r, temb, W2)


def _tcc_body(msg_ref, b2_ref, g2_ref, be2_ref, out_ref):
    z = jax.nn.relu(msg_ref[...] + b2_ref[...])
    out_ref[...] = _ln(z, g2_ref[...], be2_ref[...])


@jax.jit
def _tc_c(msg2, b2r, g2r, be2r):
    return pl.pallas_call(
        _tcc_body,
        grid=(NP // 1024,),
        in_specs=[
            pl.BlockSpec((1024, C), lambda i: (i, 0)),
            pl.BlockSpec((1, C), lambda i: (0, 0)),
            pl.BlockSpec((1, C), lambda i: (0, 0)),
            pl.BlockSpec((1, C), lambda i: (0, 0)),
        ],
        out_specs=pl.BlockSpec((1024, C), lambda i: (i, 0)),
        out_shape=jax.ShapeDtypeStruct((NP, C), jnp.float32),
    )(msg2, b2r, g2r, be2r)


def kernel(x, edge_index, edge_weight, t, W1, b1, g1, be1, W2, b2, g2, be2,
           Wt, bt):
    xp = jnp.pad(x, ((0, NP - N), (0, 0)))
    rows_s, cols_s, ews_s, counts, deg = _sc_partition(
        edge_index[0], edge_index[1], edge_weight)
    y1, dis2d, temb = _tc_a(xp, W1, deg.reshape(NP // 128, 128),
                            t.reshape(1, C), Wt, bt.reshape(1, C))
    dis = dis2d.reshape(NP)
    msg1 = _sc_conv(y1, rows_s, cols_s, ews_s, counts, dis)
    y2 = _tc_b(msg1, b1.reshape(1, C), g1.reshape(1, C), be1.reshape(1, C),
               temb, W2)
    msg2 = _sc_conv(y2, rows_s, cols_s, ews_s, counts, dis)
    out = _tc_c(msg2, b2.reshape(1, C), g2.reshape(1, C), be2.reshape(1, C))
    return out[:N]


# speedup vs baseline: 5.1212x; 1.1453x over previous
import jax
import jax.numpy as jnp
from jax import lax
from jax.experimental import pallas as pl
from jax.experimental.pallas import tpu as pltpu
from jax.experimental.pallas import tpu_sc as plsc

N = 10000
E = 160000
C = 256
NP = 10240
NT = 32
NPT = NP // NT
L = 16

CHUNK = 2000
NCHUNK = E // CHUNK
FLUSH = 2048
CB = FLUSH + 80
CAP = 160320

MCH = 480
GB = 48
NB = MCH // GB


def _mesh():
    return plsc.VectorSubcoreMesh(core_axis_name="c", subcore_axis_name="s")


def _wid():
    return lax.axis_index("s") * 2 + lax.axis_index("c")


def _partition_body(rowe_hbm, cole_hbm, ew_hbm, rows_hbm, cols_hbm,
                    ews_hbm, cnt_hbm, deg_hbm, row_in0, row_in1,
                    col_in0, col_in1, ew_in0, ew_in1,
                    row_cb, col_cb, ew_cb, deg_ln, deg_out, cnt_v,
                    fill_sm, off_sm, sem0, sem1):
    wid = _wid()
    base = wid * NPT
    lanes = lax.iota(jnp.int32, L)

    zf = jnp.zeros((L,), jnp.float32)

    def zbody(j, _):
        deg_ln[pl.ds(j * L, L)] = zf
        return 0
    lax.fori_loop(0, (L * NPT) // L, zbody, 0)
    fill_sm[0] = 0
    off_sm[0] = 0

    bufs = ((row_in0, col_in0, ew_in0, sem0), (row_in1, col_in1, ew_in1, sem1))

    def issue(c, s):
        o = c * CHUNK
        ri, ci, wi, sem = bufs[s]
        pltpu.async_copy(rowe_hbm.at[pl.ds(o, CHUNK)], ri, sem)
        pltpu.async_copy(cole_hbm.at[pl.ds(o, CHUNK)], ci, sem)
        pltpu.async_copy(ew_hbm.at[pl.ds(o, CHUNK)], wi, sem)

    def wait(c, s):
        o = c * CHUNK
        ri, ci, wi, sem = bufs[s]
        pltpu.make_async_copy(rowe_hbm.at[pl.ds(o, CHUNK)], ri, sem).wait()
        pltpu.make_async_copy(cole_hbm.at[pl.ds(o, CHUNK)], ci, sem).wait()
        pltpu.make_async_copy(ew_hbm.at[pl.ds(o, CHUNK)], wi, sem).wait()

    issue(0, 0)

    def chunk_body(c, _):
        slot = lax.rem(c, 2)

        @pl.when(c + 1 < NCHUNK)
        def _():
            lax.cond(slot == 0, lambda: issue(c + 1, 1),
                     lambda: issue(c + 1, 0))

        lax.cond(slot == 0, lambda: wait(c, 0), lambda: wait(c, 1))

        def pr(s):
            ri, ci, wi, _sem = bufs[s]

            def gbody(g, _):
                col16 = ci[pl.ds(g * L, L)]
                row16 = ri[pl.ds(g * L, L)]
                ew16 = wi[pl.ds(g * L, L)]
                m = (col16 >= base) & (col16 < base + NPT)
                cl = jnp.where(m, col16 - base, 0)
                plsc.addupdate_scatter(deg_ln, [lanes * NPT + cl], ew16, mask=m)
                fill = fill_sm[0]
                plsc.store_compressed(row_cb.at[pl.ds(fill, L)], row16, mask=m)
                plsc.store_compressed(col_cb.at[pl.ds(fill, L)], col16, mask=m)
                plsc.store_compressed(ew_cb.at[pl.ds(fill, L)], ew16, mask=m)
                cnt = jnp.sum(m.astype(jnp.int32))
                fill = fill + cnt
                fill_sm[0] = fill

                @pl.when(fill >= FLUSH)
                def _():
                    oo = off_sm[0]
                    pltpu.sync_copy(row_cb.at[pl.ds(0, FLUSH)],
                                    rows_hbm.at[pl.ds(pl.multiple_of(wid * CAP + oo, 64), FLUSH)])
                    pltpu.sync_copy(col_cb.at[pl.ds(0, FLUSH)],
                                    cols_hbm.at[pl.ds(pl.multiple_of(wid * CAP + oo, 64), FLUSH)])
                    pltpu.sync_copy(ew_cb.at[pl.ds(0, FLUSH)],
                                    ews_hbm.at[pl.ds(pl.multiple_of(wid * CAP + oo, 64), FLUSH)])
                    row_cb[pl.ds(0, L)] = row_cb[pl.ds(FLUSH, L)]
                    col_cb[pl.ds(0, L)] = col_cb[pl.ds(FLUSH, L)]
                    ew_cb[pl.ds(0, L)] = ew_cb[pl.ds(FLUSH, L)]
                    fill_sm[0] = fill - FLUSH
                    off_sm[0] = oo + FLUSH
                return 0

            lax.fori_loop(0, CHUNK // L, gbody, 0)

        lax.cond(slot == 0, lambda: pr(0), lambda: pr(1))
        return 0

    lax.fori_loop(0, NCHUNK, chunk_body, 0)

    fill = fill_sm[0]
    pad = lax.rem(64 - lax.rem(fill, 64), 64)
    zi = jnp.zeros((L,), jnp.int32)
    bv = jnp.full((L,), 1, jnp.int32) * base
    for k in range(4):
        row_cb[pl.ds(fill + k * L, L)] = zi
        col_cb[pl.ds(fill + k * L, L)] = bv
        ew_cb[pl.ds(fill + k * L, L)] = zf
    fill = fill + pad

    def fbody(i, _):
        oo = off_sm[0]
        pltpu.sync_copy(row_cb.at[pl.ds(i * 64, 64)],
                        rows_hbm.at[pl.ds(pl.multiple_of(wid * CAP + oo + i * 64, 64), 64)])
        pltpu.sync_copy(col_cb.at[pl.ds(i * 64, 64)],
                        cols_hbm.at[pl.ds(pl.multiple_of(wid * CAP + oo + i * 64, 64), 64)])
        pltpu.sync_copy(ew_cb.at[pl.ds(i * 64, 64)],
                        ews_hbm.at[pl.ds(pl.multiple_of(wid * CAP + oo + i * 64, 64), 64)])
        return 0
    lax.fori_loop(0, fill // 64, fbody, 0)
    total = off_sm[0] + fill

    cnt_v[...] = jnp.broadcast_to(total, (L,)).astype(jnp.int32)
    pltpu.sync_copy(cnt_v, cnt_hbm.at[pl.ds(pl.multiple_of(wid * L, L), L)])

    for j in range(NPT // L):
        s = jnp.full((L,), 2.0, jnp.float32)
        for l in range(L):
            s = s + deg_ln[pl.ds(l * NPT + j * L, L)]
        deg_out[pl.ds(j * L, L)] = s
    pltpu.sync_copy(deg_out, deg_hbm.at[pl.ds(pl.multiple_of(base, 64), NPT)])


@jax.jit
def _sc_partition(row_e, col_e, edge_weight):
    f = pl.kernel(
        _partition_body,
        out_type=[
            jax.ShapeDtypeStruct((NT * CAP,), jnp.int32),
            jax.ShapeDtypeStruct((NT * CAP,), jnp.int32),
            jax.ShapeDtypeStruct((NT * CAP,), jnp.float32),
            jax.ShapeDtypeStruct((NT * L,), jnp.int32),
            jax.ShapeDtypeStruct((NP,), jnp.float32),
        ],
        mesh=_mesh(),
        compiler_params=pltpu.CompilerParams(needs_layout_passes=False),
        scratch_types=[
            pltpu.VMEM((CHUNK,), jnp.int32),
            pltpu.VMEM((CHUNK,), jnp.int32),
            pltpu.VMEM((CHUNK,), jnp.int32),
            pltpu.VMEM((CHUNK,), jnp.int32),
            pltpu.VMEM((CHUNK,), jnp.float32),
            pltpu.VMEM((CHUNK,), jnp.float32),
            pltpu.VMEM((CB,), jnp.int32),
            pltpu.VMEM((CB,), jnp.int32),
            pltpu.VMEM((CB,), jnp.float32),
            pltpu.VMEM((L * NPT,), jnp.float32),
            pltpu.VMEM((NPT,), jnp.float32),
            pltpu.VMEM((L,), jnp.int32),
            pltpu.SMEM((1,), jnp.int32),
            pltpu.SMEM((1,), jnp.int32),
            pltpu.SemaphoreType.DMA,
            pltpu.SemaphoreType.DMA,
        ],
    )
    return f(row_e, col_e, edge_weight)


def _conv_body(y_hbm, rows_hbm, cols_hbm, ews_hbm, cnt_hbm, dis_hbm,
               msg_hbm, dis_v, cnt_v, row_m, col_m, ew_m, coeff_v, stage,
               acc, msem, gsem0, gsem1, gsem2):
    wid = _wid()
    base = wid * NPT
    lanes = lax.iota(jnp.int32, L)
    gsems = (gsem0, gsem1, gsem2)

    pltpu.sync_copy(dis_hbm.at[pl.ds(pl.multiple_of(base, 64), NPT)], dis_v)
    pltpu.sync_copy(cnt_hbm.at[pl.ds(pl.multiple_of(wid * L, L), L)], cnt_v)
    total = cnt_v[...][0]

    zf = jnp.zeros((L,), jnp.float32)

    def zbody(j, _):
        for k in range(C // L):
            acc[j, pl.ds(k * L, L)] = zf
        return 0
    lax.fori_loop(0, NPT, zbody, 0)

    def selfb(bb, _):
        pltpu.sync_copy(y_hbm.at[pl.ds(pl.multiple_of(base + bb * 32, 8), 32)],
                        stage.at[pl.ds(0, 32)])

        def sbody(g, _):
            d16 = dis_v[pl.ds(bb * 32 + g * L, L)]
            c16 = 2.0 * d16
            for e in range(L):
                n = g * L + e
                cvec = jnp.broadcast_to(c16[e], (L,))
                vals = [stage[n, pl.ds(j * L, L)] for j in range(C // L)]
                for j in range(C // L):
                    plsc.addupdate(acc.at[bb * 32 + n, pl.ds(j * L, L)],
                                   cvec * vals[j])
            return 0
        lax.fori_loop(0, 2, sbody, 0)
        return 0
    lax.fori_loop(0, NPT // 32, selfb, 0)

    nchunks = lax.div(total + (MCH - 1), MCH)

    def m_issue(ci):
        mo = pl.multiple_of(lax.rem(ci, 2) * MCH, 16)
        co = pl.multiple_of(wid * CAP + ci * MCH, 16)
        pltpu.async_copy(rows_hbm.at[pl.ds(co, MCH)], row_m.at[pl.ds(mo, MCH)], msem)
        pltpu.async_copy(cols_hbm.at[pl.ds(co, MCH)], col_m.at[pl.ds(mo, MCH)], msem)
        pltpu.async_copy(ews_hbm.at[pl.ds(co, MCH)], ew_m.at[pl.ds(mo, MCH)], msem)

    def m_wait(ci):
        mo = pl.multiple_of(lax.rem(ci, 2) * MCH, 16)
        co = pl.multiple_of(wid * CAP + ci * MCH, 16)
        pltpu.make_async_copy(rows_hbm.at[pl.ds(co, MCH)],
                              row_m.at[pl.ds(mo, MCH)], msem).wait()
        pltpu.make_async_copy(cols_hbm.at[pl.ds(co, MCH)],
                              col_m.at[pl.ds(mo, MCH)], msem).wait()
        pltpu.make_async_copy(ews_hbm.at[pl.ds(co, MCH)],
                              ew_m.at[pl.ds(mo, MCH)], msem).wait()

    def coeff_pass(ci):
        mo = lax.rem(ci, 2) * MCH
        co = ci * MCH

        def coefb(g, _):
            gi = co + g * L + lanes
            m = gi < total
            r16 = jnp.where(m, row_m[pl.ds(mo + g * L, L)], 0)
            c16 = jnp.where(m, col_m[pl.ds(mo + g * L, L)], base)
            w16 = jnp.where(m, ew_m[pl.ds(mo + g * L, L)], 0.0)
            dc = plsc.load_gather(dis_v, [c16 - base])
            row_m[pl.ds(mo + g * L, L)] = r16
            col_m[pl.ds(mo + g * L, L)] = c16
            coeff_v[pl.ds(mo + g * L, L)] = w16 * dc
            return 0
        lax.fori_loop(0, MCH // L, coefb, 0)

    def g_issue(ci, b):
        mo = lax.rem(ci, 2) * MCH
        gs = lax.rem(b, 3)
        go = pl.multiple_of(gs * GB, 8)
        idx = row_m.at[pl.ds(mo + b * GB, GB)]

        def go_(k):
            pltpu.async_copy(y_hbm.at[idx], stage.at[pl.ds(go, GB)], gsems[k])
        lax.cond(gs == 0, lambda: go_(0),
                 lambda: lax.cond(gs == 1, lambda: go_(1), lambda: go_(2)))

    def g_wait(ci, b):
        mo = lax.rem(ci, 2) * MCH
        gs = lax.rem(b, 3)
        go = pl.multiple_of(gs * GB, 8)
        idx = row_m.at[pl.ds(mo + b * GB, GB)]

        def gw_(k):
            pltpu.make_async_copy(y_hbm.at[idx], stage.at[pl.ds(go, GB)],
                                  gsems[k]).wait()
        lax.cond(gs == 0, lambda: gw_(0),
                 lambda: lax.cond(gs == 1, lambda: gw_(1), lambda: gw_(2)))

    @pl.when(nchunks > 0)
    def _():
        m_issue(0)
        m_wait(0)
        coeff_pass(0)
        g_issue(0, 0)
        g_issue(0, 1)

    def chunk_body(ci, _):
        @pl.when(ci + 1 < nchunks)
        def _():
            m_issue(ci + 1)

        def batch_body(b, _):
            g_wait(ci, b)

            @pl.when(b + 2 < NB)
            def _():
                g_issue(ci, b + 2)

            mo = lax.rem(ci, 2) * MCH
            go = lax.rem(b, 3) * GB

            def ebody(g):
                i0 = mo + b * GB + g * L
                c16 = coeff_v[pl.ds(i0, L)]
                cl16 = col_m[pl.ds(i0, L)] - base
                for e in range(L):
                    cvec = jnp.broadcast_to(c16[e], (L,))
                    cl = cl16[e]
                    vals = [stage[go + g * L + e, pl.ds(j * L, L)]
                            for j in range(C // L)]
                    for j in range(C // L):
                        plsc.addupdate(acc.at[cl, pl.ds(j * L, L)],
                                       cvec * vals[j])
            plsc.parallel_loop(0, GB // L)(ebody)
            return 0
        lax.fori_loop(0, NB, batch_body, 0)

        @pl.when(ci + 1 < nchunks)
        def _():
            m_wait(ci + 1)
            coeff_pass(ci + 1)
            g_issue(ci + 1, 0)
            g_issue(ci + 1, 1)
        return 0

    lax.fori_loop(0, nchunks, chunk_body, 0)

    pltpu.sync_copy(acc, msg_hbm.at[pl.ds(pl.multiple_of(base, 64), NPT)])


@jax.jit
def _sc_conv(y, rows_s, cols_s, ews_s, counts, dis):
    f = pl.kernel(
        _conv_body,
        out_type=[jax.ShapeDtypeStruct((NP, C), jnp.float32)],
        mesh=_mesh(),
        compiler_params=pltpu.CompilerParams(needs_layout_passes=False),
        scratch_types=[
            pltpu.VMEM((NPT,), jnp.float32),
            pltpu.VMEM((L,), jnp.int32),
            pltpu.VMEM((2 * MCH,), jnp.int32),
            pltpu.VMEM((2 * MCH,), jnp.int32),
            pltpu.VMEM((2 * MCH,), jnp.float32),
            pltpu.VMEM((2 * MCH,), jnp.float32),
            pltpu.VMEM((3 * GB, C), jnp.float32),
            pltpu.VMEM((NPT, C), jnp.float32),
            pltpu.SemaphoreType.DMA,
            pltpu.SemaphoreType.DMA,
            pltpu.SemaphoreType.DMA,
            pltpu.SemaphoreType.DMA,
        ],
    )
    (msg,) = f(y, rows_s, cols_s, ews_s, counts, dis)
    return msg


def _tca_body(x_ref, w_ref, deg_ref, t_ref, wt_ref, bt_ref,
              y_ref, dis_ref, temb_ref):
    d = lax.rsqrt(deg_ref[...])
    y_ref[...] = jnp.dot(x_ref[...], w_ref[...],
                         preferred_element_type=jnp.float32) * d
    dis_ref[...] = d
    temb_ref[...] = jax.nn.relu(
        jnp.dot(t_ref[...], wt_ref[...], preferred_element_type=jnp.float32)
        + bt_ref[...])


@jax.jit
def _tc_a(xp, W1, deg2d, t2, Wt, bt2):
    return pl.pallas_call(
        _tca_body,
        grid=(NP // 1024,),
        in_specs=[
            pl.BlockSpec((1024, C), lambda i: (i, 0)),
            pl.BlockSpec((C, C), lambda i: (0, 0)),
            pl.BlockSpec((1024, 1), lambda i: (i, 0)),
            pl.BlockSpec((1, C), lambda i: (0, 0)),
            pl.BlockSpec((C, C), lambda i: (0, 0)),
            pl.BlockSpec((1, C), lambda i: (0, 0)),
        ],
        out_specs=[
            pl.BlockSpec((1024, C), lambda i: (i, 0)),
            pl.BlockSpec((1024, 1), lambda i: (i, 0)),
            pl.BlockSpec((1, C), lambda i: (0, 0)),
        ],
        out_shape=[
            jax.ShapeDtypeStruct((NP, C), jnp.float32),
            jax.ShapeDtypeStruct((NP, 1), jnp.float32),
            jax.ShapeDtypeStruct((1, C), jnp.float32),
        ],
    )(xp, W1, deg2d, t2, Wt, bt2)


def _ln(z, g, b):
    mu = jnp.mean(z, axis=-1, keepdims=True)
    var = jnp.mean((z - mu) ** 2, axis=-1, keepdims=True)
    return (z - mu) * lax.rsqrt(var + 1e-5) * g + b


def _tcb_body(msg_ref, b1_ref, g1_ref, be1_ref, temb_ref, w2_ref, dis_ref,
              y2_ref):
    z = jax.nn.relu(msg_ref[...] + b1_ref[...])
    h = _ln(z, g1_ref[...], be1_ref[...]) + temb_ref[...]
    y2_ref[...] = jnp.dot(h, w2_ref[...],
                          preferred_element_type=jnp.float32) * dis_ref[...]


@jax.jit
def _tc_b(msg1, b1r, g1r, be1r, temb, W2, dis1):
    return pl.pallas_call(
        _tcb_body,
        grid=(NP // 1024,),
        in_specs=[
            pl.BlockSpec((1024, C), lambda i: (i, 0)),
            pl.BlockSpec((1, C), lambda i: (0, 0)),
            pl.BlockSpec((1, C), lambda i: (0, 0)),
            pl.BlockSpec((1, C), lambda i: (0, 0)),
            pl.BlockSpec((1, C), lambda i: (0, 0)),
            pl.BlockSpec((C, C), lambda i: (0, 0)),
            pl.BlockSpec((1024, 1), lambda i: (i, 0)),
        ],
        out_specs=pl.BlockSpec((1024, C), lambda i: (i, 0)),
        out_shape=jax.ShapeDtypeStruct((NP, C), jnp.float32),
    )(msg1, b1r, g1r, be1r, temb, W2, dis1)


def _tcc_body(msg_ref, b2_ref, g2_ref, be2_ref, out_ref):
    z = jax.nn.relu(msg_ref[...] + b2_ref[...])
    out_ref[...] = _ln(z, g2_ref[...], be2_ref[...])


@jax.jit
def _tc_c(msg2, b2r, g2r, be2r):
    return pl.pallas_call(
        _tcc_body,
        grid=(NP // 1024,),
        in_specs=[
            pl.BlockSpec((1024, C), lambda i: (i, 0)),
            pl.BlockSpec((1, C), lambda i: (0, 0)),
            pl.BlockSpec((1, C), lambda i: (0, 0)),
            pl.BlockSpec((1, C), lambda i: (0, 0)),
        ],
        out_specs=pl.BlockSpec((1024, C), lambda i: (i, 0)),
        out_shape=jax.ShapeDtypeStruct((NP, C), jnp.float32),
    )(msg2, b2r, g2r, be2r)


def kernel(x, edge_index, edge_weight, t, W1, b1, g1, be1, W2, b2, g2, be2,
           Wt, bt):
    xp = jnp.pad(x, ((0, NP - N), (0, 0)))
    rows_s, cols_s, ews_s, counts, deg = _sc_partition(
        edge_index[0], edge_index[1], edge_weight)
    y1, dis2d, temb = _tc_a(xp, W1, deg.reshape(NP, 1),
                            t.reshape(1, C), Wt, bt.reshape(1, C))
    dis = dis2d.reshape(NP)
    msg1 = _sc_conv(y1, rows_s, cols_s, ews_s, counts, dis)
    y2 = _tc_b(msg1, b1.reshape(1, C), g1.reshape(1, C), be1.reshape(1, C),
               temb, W2, dis2d)
    msg2 = _sc_conv(y2, rows_s, cols_s, ews_s, counts, dis)
    out = _tc_c(msg2, b2.reshape(1, C), g2.reshape(1, C), be2.reshape(1, C))
    return out[:N]


# speedup vs baseline: 5.3069x; 1.0363x over previous
import jax
import jax.numpy as jnp
from jax import lax
from jax.experimental import pallas as pl
from jax.experimental.pallas import tpu as pltpu
from jax.experimental.pallas import tpu_sc as plsc

N = 10000
E = 160000
C = 256
NP = 10240
NT = 32
NPT = NP // NT
L = 16

CHUNK = 2000
NCHUNK = E // CHUNK
FLUSH = 2048
CB = FLUSH + 80
CAP = 160320

MCH = 320
GB = 64
NB = MCH // GB
DEPTH = 2


def _mesh():
    return plsc.VectorSubcoreMesh(core_axis_name="c", subcore_axis_name="s")


def _wid():
    return lax.axis_index("s") * 2 + lax.axis_index("c")


def _partition_body(rowe_hbm, cole_hbm, ew_hbm, rows_hbm, cols_hbm,
                    ews_hbm, cnt_hbm, deg_hbm, row_in0, row_in1,
                    col_in0, col_in1, ew_in0, ew_in1,
                    row_cb, col_cb, ew_cb, deg_ln, deg_out, cnt_v,
                    fill_sm, off_sm, sem0, sem1):
    wid = _wid()
    base = wid * NPT
    lanes = lax.iota(jnp.int32, L)

    zf = jnp.zeros((L,), jnp.float32)

    def zbody(j, _):
        deg_ln[pl.ds(j * L, L)] = zf
        return 0
    lax.fori_loop(0, (L * NPT) // L, zbody, 0)
    fill_sm[0] = 0
    off_sm[0] = 0

    bufs = ((row_in0, col_in0, ew_in0, sem0), (row_in1, col_in1, ew_in1, sem1))

    def issue(c, s):
        o = c * CHUNK
        ri, ci, wi, sem = bufs[s]
        pltpu.async_copy(rowe_hbm.at[pl.ds(o, CHUNK)], ri, sem)
        pltpu.async_copy(cole_hbm.at[pl.ds(o, CHUNK)], ci, sem)
        pltpu.async_copy(ew_hbm.at[pl.ds(o, CHUNK)], wi, sem)

    def wait(c, s):
        o = c * CHUNK
        ri, ci, wi, sem = bufs[s]
        pltpu.make_async_copy(rowe_hbm.at[pl.ds(o, CHUNK)], ri, sem).wait()
        pltpu.make_async_copy(cole_hbm.at[pl.ds(o, CHUNK)], ci, sem).wait()
        pltpu.make_async_copy(ew_hbm.at[pl.ds(o, CHUNK)], wi, sem).wait()

    issue(0, 0)

    def chunk_body(c, _):
        slot = lax.rem(c, 2)

        @pl.when(c + 1 < NCHUNK)
        def _():
            lax.cond(slot == 0, lambda: issue(c + 1, 1),
                     lambda: issue(c + 1, 0))

        lax.cond(slot == 0, lambda: wait(c, 0), lambda: wait(c, 1))

        def pr(s):
            ri, ci, wi, _sem = bufs[s]

            def gbody(g, _):
                col16 = ci[pl.ds(g * L, L)]
                row16 = ri[pl.ds(g * L, L)]
                ew16 = wi[pl.ds(g * L, L)]
                m = (col16 >= base) & (col16 < base + NPT)
                cl = jnp.where(m, col16 - base, 0)
                plsc.addupdate_scatter(deg_ln, [lanes * NPT + cl], ew16, mask=m)
                fill = fill_sm[0]
                plsc.store_compressed(row_cb.at[pl.ds(fill, L)], row16, mask=m)
                plsc.store_compressed(col_cb.at[pl.ds(fill, L)], col16, mask=m)
                plsc.store_compressed(ew_cb.at[pl.ds(fill, L)], ew16, mask=m)
                cnt = jnp.sum(m.astype(jnp.int32))
                fill = fill + cnt
                fill_sm[0] = fill

                @pl.when(fill >= FLUSH)
                def _():
                    oo = off_sm[0]
                    pltpu.sync_copy(row_cb.at[pl.ds(0, FLUSH)],
                                    rows_hbm.at[pl.ds(pl.multiple_of(wid * CAP + oo, 64), FLUSH)])
                    pltpu.sync_copy(col_cb.at[pl.ds(0, FLUSH)],
                                    cols_hbm.at[pl.ds(pl.multiple_of(wid * CAP + oo, 64), FLUSH)])
                    pltpu.sync_copy(ew_cb.at[pl.ds(0, FLUSH)],
                                    ews_hbm.at[pl.ds(pl.multiple_of(wid * CAP + oo, 64), FLUSH)])
                    row_cb[pl.ds(0, L)] = row_cb[pl.ds(FLUSH, L)]
                    col_cb[pl.ds(0, L)] = col_cb[pl.ds(FLUSH, L)]
                    ew_cb[pl.ds(0, L)] = ew_cb[pl.ds(FLUSH, L)]
                    fill_sm[0] = fill - FLUSH
                    off_sm[0] = oo + FLUSH
                return 0

            lax.fori_loop(0, CHUNK // L, gbody, 0)

        lax.cond(slot == 0, lambda: pr(0), lambda: pr(1))
        return 0

    lax.fori_loop(0, NCHUNK, chunk_body, 0)

    fill = fill_sm[0]
    pad = lax.rem(64 - lax.rem(fill, 64), 64)
    zi = jnp.zeros((L,), jnp.int32)
    bv = jnp.full((L,), 1, jnp.int32) * base
    for k in range(4):
        row_cb[pl.ds(fill + k * L, L)] = zi
        col_cb[pl.ds(fill + k * L, L)] = bv
        ew_cb[pl.ds(fill + k * L, L)] = zf
    fill = fill + pad

    def fbody(i, _):
        oo = off_sm[0]
        pltpu.sync_copy(row_cb.at[pl.ds(i * 64, 64)],
                        rows_hbm.at[pl.ds(pl.multiple_of(wid * CAP + oo + i * 64, 64), 64)])
        pltpu.sync_copy(col_cb.at[pl.ds(i * 64, 64)],
                        cols_hbm.at[pl.ds(pl.multiple_of(wid * CAP + oo + i * 64, 64), 64)])
        pltpu.sync_copy(ew_cb.at[pl.ds(i * 64, 64)],
                        ews_hbm.at[pl.ds(pl.multiple_of(wid * CAP + oo + i * 64, 64), 64)])
        return 0
    lax.fori_loop(0, fill // 64, fbody, 0)
    total = off_sm[0] + fill

    cnt_v[...] = jnp.broadcast_to(total, (L,)).astype(jnp.int32)
    pltpu.sync_copy(cnt_v, cnt_hbm.at[pl.ds(pl.multiple_of(wid * L, L), L)])

    for j in range(NPT // L):
        s = jnp.full((L,), 2.0, jnp.float32)
        for l in range(L):
            s = s + deg_ln[pl.ds(l * NPT + j * L, L)]
        deg_out[pl.ds(j * L, L)] = s
    pltpu.sync_copy(deg_out, deg_hbm.at[pl.ds(pl.multiple_of(base, 64), NPT)])


@jax.jit
def _sc_partition(row_e, col_e, edge_weight):
    f = pl.kernel(
        _partition_body,
        out_type=[
            jax.ShapeDtypeStruct((NT * CAP,), jnp.int32),
            jax.ShapeDtypeStruct((NT * CAP,), jnp.int32),
            jax.ShapeDtypeStruct((NT * CAP,), jnp.float32),
            jax.ShapeDtypeStruct((NT * L,), jnp.int32),
            jax.ShapeDtypeStruct((NP,), jnp.float32),
        ],
        mesh=_mesh(),
        compiler_params=pltpu.CompilerParams(needs_layout_passes=False),
        scratch_types=[
            pltpu.VMEM((CHUNK,), jnp.int32),
            pltpu.VMEM((CHUNK,), jnp.int32),
            pltpu.VMEM((CHUNK,), jnp.int32),
            pltpu.VMEM((CHUNK,), jnp.int32),
            pltpu.VMEM((CHUNK,), jnp.float32),
            pltpu.VMEM((CHUNK,), jnp.float32),
            pltpu.VMEM((CB,), jnp.int32),
            pltpu.VMEM((CB,), jnp.int32),
            pltpu.VMEM((CB,), jnp.float32),
            pltpu.VMEM((L * NPT,), jnp.float32),
            pltpu.VMEM((NPT,), jnp.float32),
            pltpu.VMEM((L,), jnp.int32),
            pltpu.SMEM((1,), jnp.int32),
            pltpu.SMEM((1,), jnp.int32),
            pltpu.SemaphoreType.DMA,
            pltpu.SemaphoreType.DMA,
        ],
    )
    return f(row_e, col_e, edge_weight)


def _conv_body(y_hbm, rows_hbm, cols_hbm, ews_hbm, cnt_hbm, dis_hbm,
               msg_hbm, dis_v, cnt_v, row_m, col_m, ew_m, coeff_v, stage,
               acc, msem, gsem0, gsem1, gsem2):
    wid = _wid()
    base = wid * NPT
    lanes = lax.iota(jnp.int32, L)
    gsems = (gsem0, gsem1, gsem2)[:DEPTH]

    def _sel(gs, fns):
        if len(fns) == 1:
            fns[0]()
            return
        lax.cond(gs == 0, fns[0], lambda: _sel2(gs, fns[1:], 1))

    def _sel2(gs, fns, k):
        if len(fns) == 1:
            fns[0]()
            return
        lax.cond(gs == k, fns[0], lambda: _sel2(gs, fns[1:], k + 1))

    pltpu.sync_copy(dis_hbm.at[pl.ds(pl.multiple_of(base, 64), NPT)], dis_v)
    pltpu.sync_copy(cnt_hbm.at[pl.ds(pl.multiple_of(wid * L, L), L)], cnt_v)
    total = cnt_v[...][0]

    zf = jnp.zeros((L,), jnp.float32)

    def zbody(j, _):
        for k in range(C // L):
            acc[j, pl.ds(k * L, L)] = zf
        return 0
    lax.fori_loop(0, NPT, zbody, 0)

    def selfb(bb, _):
        pltpu.sync_copy(y_hbm.at[pl.ds(pl.multiple_of(base + bb * 32, 8), 32)],
                        stage.at[pl.ds(0, 32)])

        def sbody(g, _):
            d16 = dis_v[pl.ds(bb * 32 + g * L, L)]
            c16 = 2.0 * d16
            for e in range(L):
                n = g * L + e
                cvec = jnp.broadcast_to(c16[e], (L,))
                vals = [stage[n, pl.ds(j * L, L)] for j in range(C // L)]
                for j in range(C // L):
                    plsc.addupdate(acc.at[bb * 32 + n, pl.ds(j * L, L)],
                                   cvec * vals[j])
            return 0
        lax.fori_loop(0, 2, sbody, 0)
        return 0
    lax.fori_loop(0, NPT // 32, selfb, 0)

    nchunks = lax.div(total + (MCH - 1), MCH)

    def m_issue(ci):
        mo = pl.multiple_of(lax.rem(ci, 2) * MCH, 16)
        co = pl.multiple_of(wid * CAP + ci * MCH, 16)
        pltpu.async_copy(rows_hbm.at[pl.ds(co, MCH)], row_m.at[pl.ds(mo, MCH)], msem)
        pltpu.async_copy(cols_hbm.at[pl.ds(co, MCH)], col_m.at[pl.ds(mo, MCH)], msem)
        pltpu.async_copy(ews_hbm.at[pl.ds(co, MCH)], ew_m.at[pl.ds(mo, MCH)], msem)

    def m_wait(ci):
        mo = pl.multiple_of(lax.rem(ci, 2) * MCH, 16)
        co = pl.multiple_of(wid * CAP + ci * MCH, 16)
        pltpu.make_async_copy(rows_hbm.at[pl.ds(co, MCH)],
                              row_m.at[pl.ds(mo, MCH)], msem).wait()
        pltpu.make_async_copy(cols_hbm.at[pl.ds(co, MCH)],
                              col_m.at[pl.ds(mo, MCH)], msem).wait()
        pltpu.make_async_copy(ews_hbm.at[pl.ds(co, MCH)],
                              ew_m.at[pl.ds(mo, MCH)], msem).wait()

    def coeff_pass(ci):
        mo = lax.rem(ci, 2) * MCH
        co = ci * MCH

        def coefb(g, _):
            gi = co + g * L + lanes
            m = gi < total
            r16 = jnp.where(m, row_m[pl.ds(mo + g * L, L)], 0)
            c16 = jnp.where(m, col_m[pl.ds(mo + g * L, L)], base)
            w16 = jnp.where(m, ew_m[pl.ds(mo + g * L, L)], 0.0)
            dc = plsc.load_gather(dis_v, [c16 - base])
            row_m[pl.ds(mo + g * L, L)] = r16
            col_m[pl.ds(mo + g * L, L)] = c16
            coeff_v[pl.ds(mo + g * L, L)] = w16 * dc
            return 0
        lax.fori_loop(0, MCH // L, coefb, 0)

    def g_issue(ci, b):
        mo = lax.rem(ci, 2) * MCH
        gs = lax.rem(b, DEPTH)
        go = pl.multiple_of(gs * GB, 8)
        idx = row_m.at[pl.ds(mo + b * GB, GB)]

        def go_(k):
            def f():
                pltpu.async_copy(y_hbm.at[idx], stage.at[pl.ds(go, GB)],
                                 gsems[k])
                return None
            return f
        _sel(gs, [go_(k) for k in range(DEPTH)])

    def g_wait(ci, b):
        mo = lax.rem(ci, 2) * MCH
        gs = lax.rem(b, DEPTH)
        go = pl.multiple_of(gs * GB, 8)
        idx = row_m.at[pl.ds(mo + b * GB, GB)]

        def gw_(k):
            def f():
                pltpu.make_async_copy(y_hbm.at[idx], stage.at[pl.ds(go, GB)],
                                      gsems[k]).wait()
                return None
            return f
        _sel(gs, [gw_(k) for k in range(DEPTH)])

    @pl.when(nchunks > 0)
    def _():
        m_issue(0)
        m_wait(0)
        coeff_pass(0)
        for _pb in range(DEPTH - 1):
            g_issue(0, _pb)

    def chunk_body(ci, _):
        @pl.when(ci + 1 < nchunks)
        def _():
            m_issue(ci + 1)

        def batch_body(b, _):
            g_wait(ci, b)

            @pl.when(b + (DEPTH - 1) < NB)
            def _():
                g_issue(ci, b + (DEPTH - 1))

            mo = lax.rem(ci, 2) * MCH
            go = lax.rem(b, DEPTH) * GB

            def ebody(g):
                i0 = mo + b * GB + g * L
                c16 = coeff_v[pl.ds(i0, L)]
                cl16 = col_m[pl.ds(i0, L)] - base
                for e in range(L):
                    cvec = jnp.broadcast_to(c16[e], (L,))
                    cl = cl16[e]
                    vals = [stage[go + g * L + e, pl.ds(j * L, L)]
                            for j in range(C // L)]
                    for j in range(C // L):
                        plsc.addupdate(acc.at[cl, pl.ds(j * L, L)],
                                       cvec * vals[j])
            plsc.parallel_loop(0, GB // L)(ebody)
            return 0
        lax.fori_loop(0, NB, batch_body, 0)

        @pl.when(ci + 1 < nchunks)
        def _():
            m_wait(ci + 1)
            coeff_pass(ci + 1)
            for _pb in range(DEPTH - 1):
                g_issue(ci + 1, _pb)
        return 0

    lax.fori_loop(0, nchunks, chunk_body, 0)

    pltpu.sync_copy(acc, msg_hbm.at[pl.ds(pl.multiple_of(base, 64), NPT)])


@jax.jit
def _sc_conv(y, rows_s, cols_s, ews_s, counts, dis):
    f = pl.kernel(
        _conv_body,
        out_type=[jax.ShapeDtypeStruct((NP, C), jnp.float32)],
        mesh=_mesh(),
        compiler_params=pltpu.CompilerParams(needs_layout_passes=False),
        scratch_types=[
            pltpu.VMEM((NPT,), jnp.float32),
            pltpu.VMEM((L,), jnp.int32),
            pltpu.VMEM((2 * MCH,), jnp.int32),
            pltpu.VMEM((2 * MCH,), jnp.int32),
            pltpu.VMEM((2 * MCH,), jnp.float32),
            pltpu.VMEM((2 * MCH,), jnp.float32),
            pltpu.VMEM((DEPTH * GB, C), jnp.float32),
            pltpu.VMEM((NPT, C), jnp.float32),
            pltpu.SemaphoreType.DMA,
            pltpu.SemaphoreType.DMA,
            pltpu.SemaphoreType.DMA,
            pltpu.SemaphoreType.DMA,
        ],
    )
    (msg,) = f(y, rows_s, cols_s, ews_s, counts, dis)
    return msg


def _tca_body(x_ref, w_ref, deg_ref, t_ref, wt_ref, bt_ref,
              y_ref, dis_ref, temb_ref):
    d = lax.rsqrt(deg_ref[...])
    y_ref[...] = jnp.dot(x_ref[...], w_ref[...],
                         preferred_element_type=jnp.float32) * d
    dis_ref[...] = d
    temb_ref[...] = jax.nn.relu(
        jnp.dot(t_ref[...], wt_ref[...], preferred_element_type=jnp.float32)
        + bt_ref[...])


@jax.jit
def _tc_a(xp, W1, deg2d, t2, Wt, bt2):
    return pl.pallas_call(
        _tca_body,
        grid=(NP // 1024,),
        in_specs=[
            pl.BlockSpec((1024, C), lambda i: (i, 0)),
            pl.BlockSpec((C, C), lambda i: (0, 0)),
            pl.BlockSpec((1024, 1), lambda i: (i, 0)),
            pl.BlockSpec((1, C), lambda i: (0, 0)),
            pl.BlockSpec((C, C), lambda i: (0, 0)),
            pl.BlockSpec((1, C), lambda i: (0, 0)),
        ],
        out_specs=[
            pl.BlockSpec((1024, C), lambda i: (i, 0)),
            pl.BlockSpec((1024, 1), lambda i: (i, 0)),
            pl.BlockSpec((1, C), lambda i: (0, 0)),
        ],
        out_shape=[
            jax.ShapeDtypeStruct((NP, C), jnp.float32),
            jax.ShapeDtypeStruct((NP, 1), jnp.float32),
            jax.ShapeDtypeStruct((1, C), jnp.float32),
        ],
    )(xp, W1, deg2d, t2, Wt, bt2)


def _ln(z, g, b):
    mu = jnp.mean(z, axis=-1, keepdims=True)
    var = jnp.mean((z - mu) ** 2, axis=-1, keepdims=True)
    return (z - mu) * lax.rsqrt(var + 1e-5) * g + b


def _tcb_body(msg_ref, b1_ref, g1_ref, be1_ref, temb_ref, w2_ref, dis_ref,
              y2_ref):
    z = jax.nn.relu(msg_ref[...] + b1_ref[...])
    h = _ln(z, g1_ref[...], be1_ref[...]) + temb_ref[...]
    y2_ref[...] = jnp.dot(h, w2_ref[...],
                          preferred_element_type=jnp.float32) * dis_ref[...]


@jax.jit
def _tc_b(msg1, b1r, g1r, be1r, temb, W2, dis1):
    return pl.pallas_call(
        _tcb_body,
        grid=(NP // 1024,),
        in_specs=[
            pl.BlockSpec((1024, C), lambda i: (i, 0)),
            pl.BlockSpec((1, C), lambda i: (0, 0)),
            pl.BlockSpec((1, C), lambda i: (0, 0)),
            pl.BlockSpec((1, C), lambda i: (0, 0)),
            pl.BlockSpec((1, C), lambda i: (0, 0)),
            pl.BlockSpec((C, C), lambda i: (0, 0)),
            pl.BlockSpec((1024, 1), lambda i: (i, 0)),
        ],
        out_specs=pl.BlockSpec((1024, C), lambda i: (i, 0)),
        out_shape=jax.ShapeDtypeStruct((NP, C), jnp.float32),
    )(msg1, b1r, g1r, be1r, temb, W2, dis1)


def _tcc_body(msg_ref, b2_ref, g2_ref, be2_ref, out_ref):
    z = jax.nn.relu(msg_ref[...] + b2_ref[...])
    out_ref[...] = _ln(z, g2_ref[...], be2_ref[...])


@jax.jit
def _tc_c(msg2, b2r, g2r, be2r):
    return pl.pallas_call(
        _tcc_body,
        grid=(NP // 1024,),
        in_specs=[
            pl.BlockSpec((1024, C), lambda i: (i, 0)),
            pl.BlockSpec((1, C), lambda i: (0, 0)),
            pl.BlockSpec((1, C), lambda i: (0, 0)),
            pl.BlockSpec((1, C), lambda i: (0, 0)),
        ],
        out_specs=pl.BlockSpec((1024, C), lambda i: (i, 0)),
        out_shape=jax.ShapeDtypeStruct((NP, C), jnp.float32),
    )(msg2, b2r, g2r, be2r)


def kernel(x, edge_index, edge_weight, t, W1, b1, g1, be1, W2, b2, g2, be2,
           Wt, bt):
    xp = jnp.pad(x, ((0, NP - N), (0, 0)))
    rows_s, cols_s, ews_s, counts, deg = _sc_partition(
        edge_index[0], edge_index[1], edge_weight)
    y1, dis2d, temb = _tc_a(xp, W1, deg.reshape(NP, 1),
                            t.reshape(1, C), Wt, bt.reshape(1, C))
    dis = dis2d.reshape(NP)
    msg1 = _sc_conv(y1, rows_s, cols_s, ews_s, counts, dis)
    y2 = _tc_b(msg1, b1.reshape(1, C), g1.reshape(1, C), be1.reshape(1, C),
               temb, W2, dis2d)
    msg2 = _sc_conv(y2, rows_s, cols_s, ews_s, counts, dis)
    out = _tc_c(msg2, b2.reshape(1, C), g2.reshape(1, C), be2.reshape(1, C))
    return out[:N]


# speedup vs baseline: 7.5048x; 1.4142x over previous
import jax
import jax.numpy as jnp
from jax import lax
from jax.experimental import pallas as pl
from jax.experimental.pallas import tpu as pltpu
from jax.experimental.pallas import tpu_sc as plsc

N = 10000
E = 160000
C = 256
NP = 10240
NT = 32
NPT = NP // NT
L = 16

CHUNK = 2000
NCHUNK = E // CHUNK
FLUSH = 2048
CB = FLUSH + 2080
CAP = 162240

MCH = 320
GB = 64
NB = MCH // GB
DEPTH = 3
CW = C // 2


def _mesh():
    return plsc.VectorSubcoreMesh(core_axis_name="c", subcore_axis_name="s")


def _wid():
    return lax.axis_index("s") * 2 + lax.axis_index("c")


def _partition_body(rowe_hbm, cole_hbm, ew_hbm, rows_hbm, cols_hbm,
                    ews_hbm, cnt_hbm, deg_hbm, row_in0, row_in1,
                    col_in0, col_in1, ew_in0, ew_in1,
                    row_cb, col_cb, ew_cb, deg_ln, deg_out, cnt_v,
                    fill_sm, off_sm, sem0, sem1):
    wid = _wid()
    base = wid * NPT
    lanes = lax.iota(jnp.int32, L)
    zf = jnp.zeros((L,), jnp.float32)
    off_sm[0] = 0

    bufs = ((row_in0, col_in0, ew_in0, sem0), (row_in1, col_in1, ew_in1, sem1))

    def issue(c, s):
        o = c * CHUNK
        ri, ci, wi, sem = bufs[s]
        pltpu.async_copy(rowe_hbm.at[pl.ds(o, CHUNK)], ri, sem)
        pltpu.async_copy(cole_hbm.at[pl.ds(o, CHUNK)], ci, sem)
        pltpu.async_copy(ew_hbm.at[pl.ds(o, CHUNK)], wi, sem)

    def wait(c, s):
        o = c * CHUNK
        ri, ci, wi, sem = bufs[s]
        pltpu.make_async_copy(rowe_hbm.at[pl.ds(o, CHUNK)], ri, sem).wait()
        pltpu.make_async_copy(cole_hbm.at[pl.ds(o, CHUNK)], ci, sem).wait()
        pltpu.make_async_copy(ew_hbm.at[pl.ds(o, CHUNK)], wi, sem).wait()

    issue(0, 0)

    def chunk_body(c, fill):
        slot = lax.rem(c, 2)

        @pl.when(c + 1 < NCHUNK)
        def _():
            lax.cond(slot == 0, lambda: issue(c + 1, 1),
                     lambda: issue(c + 1, 0))

        lax.cond(slot == 0, lambda: wait(c, 0), lambda: wait(c, 1))

        def pr(s):
            ri, ci, wi, _sem = bufs[s]

            def gbody(g, fl):
                col16 = ci[pl.ds(g * L, L)]
                row16 = ri[pl.ds(g * L, L)]
                ew16 = wi[pl.ds(g * L, L)]
                m = (col16 >= base) & (col16 < base + NPT)
                plsc.store_compressed(row_cb.at[pl.ds(fl, L)], row16, mask=m)
                plsc.store_compressed(col_cb.at[pl.ds(fl, L)], col16, mask=m)
                plsc.store_compressed(ew_cb.at[pl.ds(fl, L)], ew16, mask=m)
                return fl + jnp.sum(m.astype(jnp.int32))
            return plsc.parallel_loop(0, CHUNK // L, carry=fill)(gbody)

        fill = lax.cond(slot == 0, lambda: pr(0), lambda: pr(1))

        @pl.when(fill >= FLUSH)
        def _():
            oo = off_sm[0]
            pltpu.sync_copy(row_cb.at[pl.ds(0, FLUSH)],
                            rows_hbm.at[pl.ds(pl.multiple_of(wid * CAP + oo, 64), FLUSH)])
            pltpu.sync_copy(col_cb.at[pl.ds(0, FLUSH)],
                            cols_hbm.at[pl.ds(pl.multiple_of(wid * CAP + oo, 64), FLUSH)])
            pltpu.sync_copy(ew_cb.at[pl.ds(0, FLUSH)],
                            ews_hbm.at[pl.ds(pl.multiple_of(wid * CAP + oo, 64), FLUSH)])
            r = fill - FLUSH

            def mv(k, _):
                row_cb[pl.ds(k * L, L)] = row_cb[pl.ds(FLUSH + k * L, L)]
                col_cb[pl.ds(k * L, L)] = col_cb[pl.ds(FLUSH + k * L, L)]
                ew_cb[pl.ds(k * L, L)] = ew_cb[pl.ds(FLUSH + k * L, L)]
                return 0
            lax.fori_loop(0, (r + L - 1) // L, mv, 0)
            off_sm[0] = oo + FLUSH
        return jnp.where(fill >= FLUSH, fill - FLUSH, fill)

    fill = lax.fori_loop(0, NCHUNK, chunk_body, 0)

    pad = lax.rem(64 - lax.rem(fill, 64), 64)
    zi = jnp.zeros((L,), jnp.int32)
    bv = jnp.full((L,), 1, jnp.int32) * base
    for k in range(4):
        row_cb[pl.ds(fill + k * L, L)] = zi
        col_cb[pl.ds(fill + k * L, L)] = bv
        ew_cb[pl.ds(fill + k * L, L)] = zf
    fill = fill + pad

    def fbody(i, _):
        oo = off_sm[0]
        pltpu.sync_copy(row_cb.at[pl.ds(i * 64, 64)],
                        rows_hbm.at[pl.ds(pl.multiple_of(wid * CAP + oo + i * 64, 64), 64)])
        pltpu.sync_copy(col_cb.at[pl.ds(i * 64, 64)],
                        cols_hbm.at[pl.ds(pl.multiple_of(wid * CAP + oo + i * 64, 64), 64)])
        pltpu.sync_copy(ew_cb.at[pl.ds(i * 64, 64)],
                        ews_hbm.at[pl.ds(pl.multiple_of(wid * CAP + oo + i * 64, 64), 64)])
        return 0
    lax.fori_loop(0, fill // 64, fbody, 0)
    total = off_sm[0] + fill

    cnt_v[...] = jnp.broadcast_to(total, (L,)).astype(jnp.int32)
    pltpu.sync_copy(cnt_v, cnt_hbm.at[pl.ds(pl.multiple_of(wid * L, L), L)])

    def dz(j, _):
        deg_ln[pl.ds(j * L, L)] = zf
        return 0
    lax.fori_loop(0, (L * NPT) // L, dz, 0)

    ndc = lax.div(total + (CHUNK - 1), CHUNK)

    def dchunk(c, _):
        co = pl.multiple_of(wid * CAP + c * CHUNK, 16)
        pltpu.sync_copy(cols_hbm.at[pl.ds(co, CHUNK)], col_in0)
        pltpu.sync_copy(ews_hbm.at[pl.ds(co, CHUNK)], ew_in0)

        def dg(g, _):
            gi = c * CHUNK + g * L + lanes
            valid = gi < total
            cl = col_in0[pl.ds(g * L, L)] - base
            cl = jnp.minimum(jnp.maximum(cl, 0), NPT - 1)
            w16 = jnp.where(valid, ew_in0[pl.ds(g * L, L)], 0.0)
            plsc.addupdate_scatter(deg_ln, [lanes * NPT + cl], w16)
            return 0
        lax.fori_loop(0, CHUNK // L, dg, 0)
        return 0
    lax.fori_loop(0, ndc, dchunk, 0)

    for j in range(NPT // L):
        s = jnp.full((L,), 2.0, jnp.float32)
        for l in range(L):
            s = s + deg_ln[pl.ds(l * NPT + j * L, L)]
        deg_out[pl.ds(j * L, L)] = s
    pltpu.sync_copy(deg_out, deg_hbm.at[pl.ds(pl.multiple_of(base, 64), NPT)])


@jax.jit
def _sc_partition(row_e, col_e, edge_weight):
    f = pl.kernel(
        _partition_body,
        out_type=[
            jax.ShapeDtypeStruct((NT * CAP,), jnp.int32),
            jax.ShapeDtypeStruct((NT * CAP,), jnp.int32),
            jax.ShapeDtypeStruct((NT * CAP,), jnp.float32),
            jax.ShapeDtypeStruct((NT * L,), jnp.int32),
            jax.ShapeDtypeStruct((NP,), jnp.float32),
        ],
        mesh=_mesh(),
        compiler_params=pltpu.CompilerParams(needs_layout_passes=False),
        scratch_types=[
            pltpu.VMEM((CHUNK,), jnp.int32),
            pltpu.VMEM((CHUNK,), jnp.int32),
            pltpu.VMEM((CHUNK,), jnp.int32),
            pltpu.VMEM((CHUNK,), jnp.int32),
            pltpu.VMEM((CHUNK,), jnp.float32),
            pltpu.VMEM((CHUNK,), jnp.float32),
            pltpu.VMEM((CB,), jnp.int32),
            pltpu.VMEM((CB,), jnp.int32),
            pltpu.VMEM((CB,), jnp.float32),
            pltpu.VMEM((L * NPT,), jnp.float32),
            pltpu.VMEM((NPT,), jnp.float32),
            pltpu.VMEM((L,), jnp.int32),
            pltpu.SMEM((1,), jnp.int32),
            pltpu.SMEM((1,), jnp.int32),
            pltpu.SemaphoreType.DMA,
            pltpu.SemaphoreType.DMA,
        ],
    )
    return f(row_e, col_e, edge_weight)


def _conv_body(y_hbm, rows_hbm, cols_hbm, ews_hbm, cnt_hbm, dis_hbm,
               msg_hbm, dis_v, cnt_v, row_m, col_m, ew_m, coeff_v, stage,
               acc, msem, gsem0, gsem1, gsem2):
    wid = _wid()
    base = wid * NPT
    lanes = lax.iota(jnp.int32, L)
    gsems = (gsem0, gsem1, gsem2)[:DEPTH]

    def _sel(gs, fns):
        if len(fns) == 1:
            fns[0]()
            return
        lax.cond(gs == 0, fns[0], lambda: _sel2(gs, fns[1:], 1))

    def _sel2(gs, fns, k):
        if len(fns) == 1:
            fns[0]()
            return
        lax.cond(gs == k, fns[0], lambda: _sel2(gs, fns[1:], k + 1))

    pltpu.sync_copy(dis_hbm.at[pl.ds(pl.multiple_of(base, 64), NPT)], dis_v)
    pltpu.sync_copy(cnt_hbm.at[pl.ds(pl.multiple_of(wid * L, L), L)], cnt_v)
    total = cnt_v[...][0]

    zf = jnp.zeros((L,), jnp.float32)

    def zbody(j, _):
        for k in range(C // L):
            acc[j, pl.ds(k * L, L)] = zf
        return 0
    lax.fori_loop(0, NPT, zbody, 0)

    mask_hi = jnp.full((L,), -65536, jnp.int32)

    def _unpack(w16):
        lo = plsc.bitcast(lax.shift_left(w16, 16), jnp.float32)
        hi = plsc.bitcast(w16 & mask_hi, jnp.float32)
        return lo, hi

    def selfb(bb, _):
        pltpu.sync_copy(y_hbm.at[pl.ds(pl.multiple_of(base + bb * 32, 8), 32)],
                        stage.at[pl.ds(0, 32)])

        def sbody(g, _):
            d16 = dis_v[pl.ds(bb * 32 + g * L, L)]
            c16 = 2.0 * d16
            for e in range(L):
                n = g * L + e
                cvec = jnp.broadcast_to(c16[e], (L,))
                ws = [stage[n, pl.ds(j * L, L)] for j in range(CW // L)]
                for j in range(CW // L):
                    lo, hi = _unpack(ws[j])
                    plsc.addupdate(acc.at[bb * 32 + n, pl.ds(2 * j * L, L)],
                                   cvec * lo)
                    plsc.addupdate(
                        acc.at[bb * 32 + n, pl.ds((2 * j + 1) * L, L)],
                        cvec * hi)
            return 0
        lax.fori_loop(0, 2, sbody, 0)
        return 0
    lax.fori_loop(0, NPT // 32, selfb, 0)

    nchunks = lax.div(total + (MCH - 1), MCH)

    def m_issue(ci):
        mo = pl.multiple_of(lax.rem(ci, 2) * MCH, 16)
        co = pl.multiple_of(wid * CAP + ci * MCH, 16)
        pltpu.async_copy(rows_hbm.at[pl.ds(co, MCH)], row_m.at[pl.ds(mo, MCH)], msem)
        pltpu.async_copy(cols_hbm.at[pl.ds(co, MCH)], col_m.at[pl.ds(mo, MCH)], msem)
        pltpu.async_copy(ews_hbm.at[pl.ds(co, MCH)], ew_m.at[pl.ds(mo, MCH)], msem)

    def m_wait(ci):
        mo = pl.multiple_of(lax.rem(ci, 2) * MCH, 16)
        co = pl.multiple_of(wid * CAP + ci * MCH, 16)
        pltpu.make_async_copy(rows_hbm.at[pl.ds(co, MCH)],
                              row_m.at[pl.ds(mo, MCH)], msem).wait()
        pltpu.make_async_copy(cols_hbm.at[pl.ds(co, MCH)],
                              col_m.at[pl.ds(mo, MCH)], msem).wait()
        pltpu.make_async_copy(ews_hbm.at[pl.ds(co, MCH)],
                              ew_m.at[pl.ds(mo, MCH)], msem).wait()

    def coeff_pass(ci):
        mo = lax.rem(ci, 2) * MCH
        co = ci * MCH

        def coefb(g, _):
            gi = co + g * L + lanes
            m = gi < total
            r16 = jnp.where(m, row_m[pl.ds(mo + g * L, L)], 0)
            c16 = jnp.where(m, col_m[pl.ds(mo + g * L, L)], base)
            w16 = jnp.where(m, ew_m[pl.ds(mo + g * L, L)], 0.0)
            dc = plsc.load_gather(dis_v, [c16 - base])
            row_m[pl.ds(mo + g * L, L)] = r16
            col_m[pl.ds(mo + g * L, L)] = c16
            coeff_v[pl.ds(mo + g * L, L)] = w16 * dc
            return 0
        lax.fori_loop(0, MCH // L, coefb, 0)

    def g_issue(ci, b):
        mo = lax.rem(ci, 2) * MCH
        gs = lax.rem(b, DEPTH)
        go = pl.multiple_of(gs * GB, 8)
        idx = row_m.at[pl.ds(mo + b * GB, GB)]

        def go_(k):
            def f():
                pltpu.async_copy(y_hbm.at[idx], stage.at[pl.ds(go, GB)],
                                 gsems[k])
                return None
            return f
        _sel(gs, [go_(k) for k in range(DEPTH)])

    def g_wait(ci, b):
        mo = lax.rem(ci, 2) * MCH
        gs = lax.rem(b, DEPTH)
        go = pl.multiple_of(gs * GB, 8)
        idx = row_m.at[pl.ds(mo + b * GB, GB)]

        def gw_(k):
            def f():
                pltpu.make_async_copy(y_hbm.at[idx], stage.at[pl.ds(go, GB)],
                                      gsems[k]).wait()
                return None
            return f
        _sel(gs, [gw_(k) for k in range(DEPTH)])

    @pl.when(nchunks > 0)
    def _():
        m_issue(0)
        m_wait(0)
        coeff_pass(0)
        for _pb in range(DEPTH - 1):
            g_issue(0, _pb)

    def chunk_body(ci, _):
        @pl.when(ci + 1 < nchunks)
        def _():
            m_issue(ci + 1)

        def batch_body(b, _):
            g_wait(ci, b)

            @pl.when(b + (DEPTH - 1) < NB)
            def _():
                g_issue(ci, b + (DEPTH - 1))

            mo = lax.rem(ci, 2) * MCH
            go = lax.rem(b, DEPTH) * GB

            def ebody(g):
                i0 = mo + b * GB + g * L
                c16 = coeff_v[pl.ds(i0, L)]
                cl16 = col_m[pl.ds(i0, L)] - base
                for e in range(L):
                    cvec = jnp.broadcast_to(c16[e], (L,))
                    cl = cl16[e]
                    ws = [stage[go + g * L + e, pl.ds(j * L, L)]
                          for j in range(CW // L)]
                    for j in range(CW // L):
                        lo, hi = _unpack(ws[j])
                        plsc.addupdate(acc.at[cl, pl.ds(2 * j * L, L)],
                                       cvec * lo)
                        plsc.addupdate(acc.at[cl, pl.ds((2 * j + 1) * L, L)],
                                       cvec * hi)
            plsc.parallel_loop(0, GB // L)(ebody)
            return 0
        lax.fori_loop(0, NB, batch_body, 0)

        @pl.when(ci + 1 < nchunks)
        def _():
            m_wait(ci + 1)
            coeff_pass(ci + 1)
            for _pb in range(DEPTH - 1):
                g_issue(ci + 1, _pb)
        return 0

    lax.fori_loop(0, nchunks, chunk_body, 0)

    pltpu.sync_copy(acc, msg_hbm.at[pl.ds(pl.multiple_of(base, 64), NPT)])


@jax.jit
def _sc_conv(y, rows_s, cols_s, ews_s, counts, dis):
    f = pl.kernel(
        _conv_body,
        out_type=[jax.ShapeDtypeStruct((NP, C), jnp.float32)],
        mesh=_mesh(),
        compiler_params=pltpu.CompilerParams(needs_layout_passes=False),
        scratch_types=[
            pltpu.VMEM((NPT,), jnp.float32),
            pltpu.VMEM((L,), jnp.int32),
            pltpu.VMEM((2 * MCH,), jnp.int32),
            pltpu.VMEM((2 * MCH,), jnp.int32),
            pltpu.VMEM((2 * MCH,), jnp.float32),
            pltpu.VMEM((2 * MCH,), jnp.float32),
            pltpu.VMEM((DEPTH * GB, CW), jnp.int32),
            pltpu.VMEM((NPT, C), jnp.float32),
            pltpu.SemaphoreType.DMA,
            pltpu.SemaphoreType.DMA,
            pltpu.SemaphoreType.DMA,
            pltpu.SemaphoreType.DMA,
        ],
    )
    (msg,) = f(y, rows_s, cols_s, ews_s, counts, dis)
    return msg


def _pack_rows(a, b):
    ai = lax.bitcast_convert_type(a, jnp.int32)
    bi = lax.bitcast_convert_type(b, jnp.int32)
    lo = lax.shift_right_logical(ai + 0x8000, 16)
    hi = (bi + 0x8000) & jnp.int32(-65536)
    return lo | hi


def _tca_body(x_ref, wa_ref, wb_ref, deg_ref, t_ref, wt_ref, bt_ref,
              y_ref, dis_ref, temb_ref):
    d = lax.rsqrt(deg_ref[...])
    a = jnp.dot(x_ref[...], wa_ref[...],
                preferred_element_type=jnp.float32) * d
    b = jnp.dot(x_ref[...], wb_ref[...],
                preferred_element_type=jnp.float32) * d
    y_ref[...] = _pack_rows(a, b)
    dis_ref[...] = d
    temb_ref[...] = jax.nn.relu(
        jnp.dot(t_ref[...], wt_ref[...], preferred_element_type=jnp.float32)
        + bt_ref[...])


@jax.jit
def _tc_a(xp, W1a, W1b, deg2d, t2, Wt, bt2):
    return pl.pallas_call(
        _tca_body,
        grid=(NP // 1024,),
        in_specs=[
            pl.BlockSpec((1024, C), lambda i: (i, 0)),
            pl.BlockSpec((C, CW), lambda i: (0, 0)),
            pl.BlockSpec((C, CW), lambda i: (0, 0)),
            pl.BlockSpec((1024, 1), lambda i: (i, 0)),
            pl.BlockSpec((1, C), lambda i: (0, 0)),
            pl.BlockSpec((C, C), lambda i: (0, 0)),
            pl.BlockSpec((1, C), lambda i: (0, 0)),
        ],
        out_specs=[
            pl.BlockSpec((1024, CW), lambda i: (i, 0)),
            pl.BlockSpec((1024, 1), lambda i: (i, 0)),
            pl.BlockSpec((1, C), lambda i: (0, 0)),
        ],
        out_shape=[
            jax.ShapeDtypeStruct((NP, CW), jnp.int32),
            jax.ShapeDtypeStruct((NP, 1), jnp.float32),
            jax.ShapeDtypeStruct((1, C), jnp.float32),
        ],
    )(xp, W1a, W1b, deg2d, t2, Wt, bt2)


def _ln(z, g, b):
    mu = jnp.mean(z, axis=-1, keepdims=True)
    var = jnp.mean((z - mu) ** 2, axis=-1, keepdims=True)
    return (z - mu) * lax.rsqrt(var + 1e-5) * g + b


def _tcb_body(msg_ref, b1_ref, g1_ref, be1_ref, temb_ref, wa_ref, wb_ref,
              dis_ref, y2_ref):
    z = jax.nn.relu(msg_ref[...] + b1_ref[...])
    h = _ln(z, g1_ref[...], be1_ref[...]) + temb_ref[...]
    d = dis_ref[...]
    a = jnp.dot(h, wa_ref[...], preferred_element_type=jnp.float32) * d
    b = jnp.dot(h, wb_ref[...], preferred_element_type=jnp.float32) * d
    y2_ref[...] = _pack_rows(a, b)


@jax.jit
def _tc_b(msg1, b1r, g1r, be1r, temb, W2a, W2b, dis1):
    return pl.pallas_call(
        _tcb_body,
        grid=(NP // 1024,),
        in_specs=[
            pl.BlockSpec((1024, C), lambda i: (i, 0)),
            pl.BlockSpec((1, C), lambda i: (0, 0)),
            pl.BlockSpec((1, C), lambda i: (0, 0)),
            pl.BlockSpec((1, C), lambda i: (0, 0)),
            pl.BlockSpec((1, C), lambda i: (0, 0)),
            pl.BlockSpec((C, CW), lambda i: (0, 0)),
            pl.BlockSpec((C, CW), lambda i: (0, 0)),
            pl.BlockSpec((1024, 1), lambda i: (i, 0)),
        ],
        out_specs=pl.BlockSpec((1024, CW), lambda i: (i, 0)),
        out_shape=jax.ShapeDtypeStruct((NP, CW), jnp.int32),
    )(msg1, b1r, g1r, be1r, temb, W2a, W2b, dis1)


def _tcc_body(msg_ref, b2_ref, g2_ref, be2_ref, out_ref):
    z = jax.nn.relu(msg_ref[...] + b2_ref[...])
    out_ref[...] = _ln(z, g2_ref[...], be2_ref[...])


@jax.jit
def _tc_c(msg2, b2r, g2r, be2r):
    return pl.pallas_call(
        _tcc_body,
        grid=(NP // 1024,),
        in_specs=[
            pl.BlockSpec((1024, C), lambda i: (i, 0)),
            pl.BlockSpec((1, C), lambda i: (0, 0)),
            pl.BlockSpec((1, C), lambda i: (0, 0)),
            pl.BlockSpec((1, C), lambda i: (0, 0)),
        ],
        out_specs=pl.BlockSpec((1024, C), lambda i: (i, 0)),
        out_shape=jax.ShapeDtypeStruct((NP, C), jnp.float32),
    )(msg2, b2r, g2r, be2r)


def kernel(x, edge_index, edge_weight, t, W1, b1, g1, be1, W2, b2, g2, be2,
           Wt, bt):
    xp = jnp.pad(x, ((0, NP - N), (0, 0)))
    ms = jnp.arange(CW, dtype=jnp.int32)
    sig_a = 32 * (ms // 16) + (ms % 16)
    sig_b = sig_a + 16
    rows_s, cols_s, ews_s, counts, deg = _sc_partition(
        edge_index[0], edge_index[1], edge_weight)
    y1, dis2d, temb = _tc_a(xp, W1[:, sig_a], W1[:, sig_b],
                            deg.reshape(NP, 1), t.reshape(1, C), Wt,
                            bt.reshape(1, C))
    dis = dis2d.reshape(NP)
    msg1 = _sc_conv(y1, rows_s, cols_s, ews_s, counts, dis)
    y2 = _tc_b(msg1, b1.reshape(1, C), g1.reshape(1, C), be1.reshape(1, C),
               temb, W2[:, sig_a], W2[:, sig_b], dis2d)
    msg2 = _sc_conv(y2, rows_s, cols_s, ews_s, counts, dis)
    out = _tc_c(msg2, b2.reshape(1, C), g2.reshape(1, C), be2.reshape(1, C))
    return out[:N]


# speedup vs baseline: 7.5517x; 1.0063x over previous
import jax
import jax.numpy as jnp
from jax import lax
from jax.experimental import pallas as pl
from jax.experimental.pallas import tpu as pltpu
from jax.experimental.pallas import tpu_sc as plsc

N = 10000
E = 160000
C = 256
NP = 10240
NT = 32
NPT = NP // NT
L = 16

CHUNK = 2000
NCHUNK = E // CHUNK
FLUSH = 2048
CB = FLUSH + 2080
CAP = 162240

MCH = 320
GB = 64
NB = MCH // GB
DEPTH = 3
CW = C // 2


def _mesh():
    return plsc.VectorSubcoreMesh(core_axis_name="c", subcore_axis_name="s")


def _wid():
    return lax.axis_index("s") * 2 + lax.axis_index("c")


def _partition_body(rowe_hbm, cole_hbm, ew_hbm, rows_hbm, cols_hbm,
                    ews_hbm, cnt_hbm, deg_hbm, row_in0, row_in1,
                    col_in0, col_in1, ew_in0, ew_in1,
                    row_cb, col_cb, ew_cb, deg_ln, deg_out, cnt_v,
                    fill_sm, off_sm, sem0, sem1):
    wid = _wid()
    base = wid * NPT
    lanes = lax.iota(jnp.int32, L)
    zf = jnp.zeros((L,), jnp.float32)
    off_sm[0] = 0

    bufs = ((row_in0, col_in0, ew_in0, sem0), (row_in1, col_in1, ew_in1, sem1))

    def issue(c, s):
        o = c * CHUNK
        ri, ci, wi, sem = bufs[s]
        pltpu.async_copy(rowe_hbm.at[pl.ds(o, CHUNK)], ri, sem)
        pltpu.async_copy(cole_hbm.at[pl.ds(o, CHUNK)], ci, sem)
        pltpu.async_copy(ew_hbm.at[pl.ds(o, CHUNK)], wi, sem)

    def wait(c, s):
        o = c * CHUNK
        ri, ci, wi, sem = bufs[s]
        pltpu.make_async_copy(rowe_hbm.at[pl.ds(o, CHUNK)], ri, sem).wait()
        pltpu.make_async_copy(cole_hbm.at[pl.ds(o, CHUNK)], ci, sem).wait()
        pltpu.make_async_copy(ew_hbm.at[pl.ds(o, CHUNK)], wi, sem).wait()

    issue(0, 0)

    def chunk_body(c, fill):
        slot = lax.rem(c, 2)

        @pl.when(c + 1 < NCHUNK)
        def _():
            lax.cond(slot == 0, lambda: issue(c + 1, 1),
                     lambda: issue(c + 1, 0))

        lax.cond(slot == 0, lambda: wait(c, 0), lambda: wait(c, 1))

        def pr(s):
            ri, ci, wi, _sem = bufs[s]

            def gbody(g, fl):
                col16 = ci[pl.ds(g * L, L)]
                row16 = ri[pl.ds(g * L, L)]
                ew16 = wi[pl.ds(g * L, L)]
                m = (col16 >= base) & (col16 < base + NPT)
                plsc.store_compressed(row_cb.at[pl.ds(fl, L)], row16, mask=m)
                plsc.store_compressed(col_cb.at[pl.ds(fl, L)], col16, mask=m)
                plsc.store_compressed(ew_cb.at[pl.ds(fl, L)], ew16, mask=m)
                return fl + jnp.sum(m.astype(jnp.int32))
            return plsc.parallel_loop(0, CHUNK // L, unroll=2,
                                       carry=fill)(gbody)

        fill = lax.cond(slot == 0, lambda: pr(0), lambda: pr(1))

        @pl.when(fill >= FLUSH)
        def _():
            oo = off_sm[0]
            pltpu.sync_copy(row_cb.at[pl.ds(0, FLUSH)],
                            rows_hbm.at[pl.ds(pl.multiple_of(wid * CAP + oo, 64), FLUSH)])
            pltpu.sync_copy(col_cb.at[pl.ds(0, FLUSH)],
                            cols_hbm.at[pl.ds(pl.multiple_of(wid * CAP + oo, 64), FLUSH)])
            pltpu.sync_copy(ew_cb.at[pl.ds(0, FLUSH)],
                            ews_hbm.at[pl.ds(pl.multiple_of(wid * CAP + oo, 64), FLUSH)])
            r = fill - FLUSH

            def mv(k, _):
                row_cb[pl.ds(k * L, L)] = row_cb[pl.ds(FLUSH + k * L, L)]
                col_cb[pl.ds(k * L, L)] = col_cb[pl.ds(FLUSH + k * L, L)]
                ew_cb[pl.ds(k * L, L)] = ew_cb[pl.ds(FLUSH + k * L, L)]
                return 0
            lax.fori_loop(0, (r + L - 1) // L, mv, 0)
            off_sm[0] = oo + FLUSH
        return jnp.where(fill >= FLUSH, fill - FLUSH, fill)

    fill = lax.fori_loop(0, NCHUNK, chunk_body, 0)

    pad = lax.rem(64 - lax.rem(fill, 64), 64)
    zi = jnp.zeros((L,), jnp.int32)
    bv = jnp.full((L,), 1, jnp.int32) * base
    for k in range(4):
        row_cb[pl.ds(fill + k * L, L)] = zi
        col_cb[pl.ds(fill + k * L, L)] = bv
        ew_cb[pl.ds(fill + k * L, L)] = zf
    fill = fill + pad

    def fbody(i, _):
        oo = off_sm[0]
        pltpu.sync_copy(row_cb.at[pl.ds(i * 64, 64)],
                        rows_hbm.at[pl.ds(pl.multiple_of(wid * CAP + oo + i * 64, 64), 64)])
        pltpu.sync_copy(col_cb.at[pl.ds(i * 64, 64)],
                        cols_hbm.at[pl.ds(pl.multiple_of(wid * CAP + oo + i * 64, 64), 64)])
        pltpu.sync_copy(ew_cb.at[pl.ds(i * 64, 64)],
                        ews_hbm.at[pl.ds(pl.multiple_of(wid * CAP + oo + i * 64, 64), 64)])
        return 0
    lax.fori_loop(0, fill // 64, fbody, 0)
    total = off_sm[0] + fill

    cnt_v[...] = jnp.broadcast_to(total, (L,)).astype(jnp.int32)
    pltpu.sync_copy(cnt_v, cnt_hbm.at[pl.ds(pl.multiple_of(wid * L, L), L)])

    def dz(j, _):
        deg_ln[pl.ds(j * L, L)] = zf
        return 0
    lax.fori_loop(0, (L * NPT) // L, dz, 0)

    ndc = lax.div(total + (CHUNK - 1), CHUNK)

    def dchunk(c, _):
        co = pl.multiple_of(wid * CAP + c * CHUNK, 16)
        pltpu.sync_copy(cols_hbm.at[pl.ds(co, CHUNK)], col_in0)
        pltpu.sync_copy(ews_hbm.at[pl.ds(co, CHUNK)], ew_in0)

        def dg(g, _):
            gi = c * CHUNK + g * L + lanes
            valid = gi < total
            cl = col_in0[pl.ds(g * L, L)] - base
            cl = jnp.minimum(jnp.maximum(cl, 0), NPT - 1)
            w16 = jnp.where(valid, ew_in0[pl.ds(g * L, L)], 0.0)
            plsc.addupdate_scatter(deg_ln, [lanes * NPT + cl], w16)
            return 0
        lax.fori_loop(0, CHUNK // L, dg, 0)
        return 0
    lax.fori_loop(0, ndc, dchunk, 0)

    for j in range(NPT // L):
        s = jnp.full((L,), 2.0, jnp.float32)
        for l in range(L):
            s = s + deg_ln[pl.ds(l * NPT + j * L, L)]
        deg_out[pl.ds(j * L, L)] = s
    pltpu.sync_copy(deg_out, deg_hbm.at[pl.ds(pl.multiple_of(base, 64), NPT)])


@jax.jit
def _sc_partition(row_e, col_e, edge_weight):
    f = pl.kernel(
        _partition_body,
        out_type=[
            jax.ShapeDtypeStruct((NT * CAP,), jnp.int32),
            jax.ShapeDtypeStruct((NT * CAP,), jnp.int32),
            jax.ShapeDtypeStruct((NT * CAP,), jnp.float32),
            jax.ShapeDtypeStruct((NT * L,), jnp.int32),
            jax.ShapeDtypeStruct((NP,), jnp.float32),
        ],
        mesh=_mesh(),
        compiler_params=pltpu.CompilerParams(needs_layout_passes=False),
        scratch_types=[
            pltpu.VMEM((CHUNK,), jnp.int32),
            pltpu.VMEM((CHUNK,), jnp.int32),
            pltpu.VMEM((CHUNK,), jnp.int32),
            pltpu.VMEM((CHUNK,), jnp.int32),
            pltpu.VMEM((CHUNK,), jnp.float32),
            pltpu.VMEM((CHUNK,), jnp.float32),
            pltpu.VMEM((CB,), jnp.int32),
            pltpu.VMEM((CB,), jnp.int32),
            pltpu.VMEM((CB,), jnp.float32),
            pltpu.VMEM((L * NPT,), jnp.float32),
            pltpu.VMEM((NPT,), jnp.float32),
            pltpu.VMEM((L,), jnp.int32),
            pltpu.SMEM((1,), jnp.int32),
            pltpu.SMEM((1,), jnp.int32),
            pltpu.SemaphoreType.DMA,
            pltpu.SemaphoreType.DMA,
        ],
    )
    return f(row_e, col_e, edge_weight)


def _conv_body(y_hbm, rows_hbm, cols_hbm, ews_hbm, cnt_hbm, dis_hbm,
               msg_hbm, dis_v, cnt_v, row_m, col_m, ew_m, coeff_v, stage,
               acc, msem, gsem0, gsem1, gsem2):
    wid = _wid()
    base = wid * NPT
    lanes = lax.iota(jnp.int32, L)
    gsems = (gsem0, gsem1, gsem2)[:DEPTH]

    def _sel(gs, fns):
        if len(fns) == 1:
            fns[0]()
            return
        lax.cond(gs == 0, fns[0], lambda: _sel2(gs, fns[1:], 1))

    def _sel2(gs, fns, k):
        if len(fns) == 1:
            fns[0]()
            return
        lax.cond(gs == k, fns[0], lambda: _sel2(gs, fns[1:], k + 1))

    pltpu.sync_copy(dis_hbm.at[pl.ds(pl.multiple_of(base, 64), NPT)], dis_v)
    pltpu.sync_copy(cnt_hbm.at[pl.ds(pl.multiple_of(wid * L, L), L)], cnt_v)
    total = cnt_v[...][0]

    zf = jnp.zeros((L,), jnp.float32)

    def zbody(j, _):
        for k in range(C // L):
            acc[j, pl.ds(k * L, L)] = zf
        return 0
    lax.fori_loop(0, NPT, zbody, 0)

    mask_hi = jnp.full((L,), -65536, jnp.int32)

    def _unpack(w16):
        lo = plsc.bitcast(lax.shift_left(w16, 16), jnp.float32)
        hi = plsc.bitcast(w16 & mask_hi, jnp.float32)
        return lo, hi

    def selfb(bb, _):
        pltpu.sync_copy(y_hbm.at[pl.ds(pl.multiple_of(base + bb * 32, 8), 32)],
                        stage.at[pl.ds(0, 32)])

        def sbody(g, _):
            d16 = dis_v[pl.ds(bb * 32 + g * L, L)]
            c16 = 2.0 * d16
            for e in range(L):
                n = g * L + e
                cvec = jnp.broadcast_to(c16[e], (L,))
                ws = [stage[n, pl.ds(j * L, L)] for j in range(CW // L)]
                for j in range(CW // L):
                    lo, hi = _unpack(ws[j])
                    plsc.addupdate(acc.at[bb * 32 + n, pl.ds(2 * j * L, L)],
                                   cvec * lo)
                    plsc.addupdate(
                        acc.at[bb * 32 + n, pl.ds((2 * j + 1) * L, L)],
                        cvec * hi)
            return 0
        lax.fori_loop(0, 2, sbody, 0)
        return 0
    lax.fori_loop(0, NPT // 32, selfb, 0)

    nchunks = lax.div(total + (MCH - 1), MCH)

    def m_issue(ci):
        mo = pl.multiple_of(lax.rem(ci, 2) * MCH, 16)
        co = pl.multiple_of(wid * CAP + ci * MCH, 16)
        pltpu.async_copy(rows_hbm.at[pl.ds(co, MCH)], row_m.at[pl.ds(mo, MCH)], msem)
        pltpu.async_copy(cols_hbm.at[pl.ds(co, MCH)], col_m.at[pl.ds(mo, MCH)], msem)
        pltpu.async_copy(ews_hbm.at[pl.ds(co, MCH)], ew_m.at[pl.ds(mo, MCH)], msem)

    def m_wait(ci):
        mo = pl.multiple_of(lax.rem(ci, 2) * MCH, 16)
        co = pl.multiple_of(wid * CAP + ci * MCH, 16)
        pltpu.make_async_copy(rows_hbm.at[pl.ds(co, MCH)],
                              row_m.at[pl.ds(mo, MCH)], msem).wait()
        pltpu.make_async_copy(cols_hbm.at[pl.ds(co, MCH)],
                              col_m.at[pl.ds(mo, MCH)], msem).wait()
        pltpu.make_async_copy(ews_hbm.at[pl.ds(co, MCH)],
                              ew_m.at[pl.ds(mo, MCH)], msem).wait()

    def coeff_pass(ci):
        mo = lax.rem(ci, 2) * MCH
        co = ci * MCH

        def coefb(g, _):
            gi = co + g * L + lanes
            m = gi < total
            r16 = jnp.where(m, row_m[pl.ds(mo + g * L, L)], 0)
            c16 = jnp.where(m, col_m[pl.ds(mo + g * L, L)], base)
            w16 = jnp.where(m, ew_m[pl.ds(mo + g * L, L)], 0.0)
            dc = plsc.load_gather(dis_v, [c16 - base])
            row_m[pl.ds(mo + g * L, L)] = r16
            col_m[pl.ds(mo + g * L, L)] = c16
            coeff_v[pl.ds(mo + g * L, L)] = w16 * dc
            return 0
        lax.fori_loop(0, MCH // L, coefb, 0)

    def g_issue(ci, b):
        mo = lax.rem(ci, 2) * MCH
        gs = lax.rem(b, DEPTH)
        go = pl.multiple_of(gs * GB, 8)
        idx = row_m.at[pl.ds(mo + b * GB, GB)]

        def go_(k):
            def f():
                pltpu.async_copy(y_hbm.at[idx], stage.at[pl.ds(go, GB)],
                                 gsems[k])
                return None
            return f
        _sel(gs, [go_(k) for k in range(DEPTH)])

    def g_wait(ci, b):
        mo = lax.rem(ci, 2) * MCH
        gs = lax.rem(b, DEPTH)
        go = pl.multiple_of(gs * GB, 8)
        idx = row_m.at[pl.ds(mo + b * GB, GB)]

        def gw_(k):
            def f():
                pltpu.make_async_copy(y_hbm.at[idx], stage.at[pl.ds(go, GB)],
                                      gsems[k]).wait()
                return None
            return f
        _sel(gs, [gw_(k) for k in range(DEPTH)])

    @pl.when(nchunks > 0)
    def _():
        m_issue(0)
        m_wait(0)
        coeff_pass(0)
        for _pb in range(DEPTH - 1):
            g_issue(0, _pb)

    def chunk_body(ci, _):
        @pl.when(ci + 1 < nchunks)
        def _():
            m_issue(ci + 1)

        def batch_body(b, _):
            g_wait(ci, b)

            @pl.when(b + (DEPTH - 1) < NB)
            def _():
                g_issue(ci, b + (DEPTH - 1))

            mo = lax.rem(ci, 2) * MCH
            go = lax.rem(b, DEPTH) * GB

            def ebody(g):
                i0 = mo + b * GB + g * L
                c16 = coeff_v[pl.ds(i0, L)]
                cl16 = col_m[pl.ds(i0, L)] - base
                for e in range(L):
                    cvec = jnp.broadcast_to(c16[e], (L,))
                    cl = cl16[e]
                    ws = [stage[go + g * L + e, pl.ds(j * L, L)]
                          for j in range(CW // L)]
                    for j in range(CW // L):
                        lo, hi = _unpack(ws[j])
                        plsc.addupdate(acc.at[cl, pl.ds(2 * j * L, L)],
                                       cvec * lo)
                        plsc.addupdate(acc.at[cl, pl.ds((2 * j + 1) * L, L)],
                                       cvec * hi)
            plsc.parallel_loop(0, GB // L)(ebody)
            return 0
        lax.fori_loop(0, NB, batch_body, 0)

        @pl.when(ci + 1 < nchunks)
        def _():
            m_wait(ci + 1)
            coeff_pass(ci + 1)
            for _pb in range(DEPTH - 1):
                g_issue(ci + 1, _pb)
        return 0

    lax.fori_loop(0, nchunks, chunk_body, 0)

    pltpu.sync_copy(acc, msg_hbm.at[pl.ds(pl.multiple_of(base, 64), NPT)])


@jax.jit
def _sc_conv(y, rows_s, cols_s, ews_s, counts, dis):
    f = pl.kernel(
        _conv_body,
        out_type=[jax.ShapeDtypeStruct((NP, C), jnp.float32)],
        mesh=_mesh(),
        compiler_params=pltpu.CompilerParams(needs_layout_passes=False),
        scratch_types=[
            pltpu.VMEM((NPT,), jnp.float32),
            pltpu.VMEM((L,), jnp.int32),
            pltpu.VMEM((2 * MCH,), jnp.int32),
            pltpu.VMEM((2 * MCH,), jnp.int32),
            pltpu.VMEM((2 * MCH,), jnp.float32),
            pltpu.VMEM((2 * MCH,), jnp.float32),
            pltpu.VMEM((DEPTH * GB, CW), jnp.int32),
            pltpu.VMEM((NPT, C), jnp.float32),
            pltpu.SemaphoreType.DMA,
            pltpu.SemaphoreType.DMA,
            pltpu.SemaphoreType.DMA,
            pltpu.SemaphoreType.DMA,
        ],
    )
    (msg,) = f(y, rows_s, cols_s, ews_s, counts, dis)
    return msg


def _pack_rows(a, b):
    ai = lax.bitcast_convert_type(a, jnp.int32)
    bi = lax.bitcast_convert_type(b, jnp.int32)
    lo = lax.shift_right_logical(ai + 0x8000, 16)
    hi = (bi + 0x8000) & jnp.int32(-65536)
    return lo | hi


def _tca_body(x_ref, wa_ref, wb_ref, deg_ref, t_ref, wt_ref, bt_ref,
              y_ref, dis_ref, temb_ref):
    d = lax.rsqrt(deg_ref[...])
    a = jnp.dot(x_ref[...], wa_ref[...],
                preferred_element_type=jnp.float32) * d
    b = jnp.dot(x_ref[...], wb_ref[...],
                preferred_element_type=jnp.float32) * d
    y_ref[...] = _pack_rows(a, b)
    dis_ref[...] = d
    temb_ref[...] = jax.nn.relu(
        jnp.dot(t_ref[...], wt_ref[...], preferred_element_type=jnp.float32)
        + bt_ref[...])


@jax.jit
def _tc_a(xp, W1a, W1b, deg2d, t2, Wt, bt2):
    return pl.pallas_call(
        _tca_body,
        grid=(NP // 1024,),
        in_specs=[
            pl.BlockSpec((1024, C), lambda i: (i, 0)),
            pl.BlockSpec((C, CW), lambda i: (0, 0)),
            pl.BlockSpec((C, CW), lambda i: (0, 0)),
            pl.BlockSpec((1024, 1), lambda i: (i, 0)),
            pl.BlockSpec((1, C), lambda i: (0, 0)),
            pl.BlockSpec((C, C), lambda i: (0, 0)),
            pl.BlockSpec((1, C), lambda i: (0, 0)),
        ],
        out_specs=[
            pl.BlockSpec((1024, CW), lambda i: (i, 0)),
            pl.BlockSpec((1024, 1), lambda i: (i, 0)),
            pl.BlockSpec((1, C), lambda i: (0, 0)),
        ],
        out_shape=[
            jax.ShapeDtypeStruct((NP, CW), jnp.int32),
            jax.ShapeDtypeStruct((NP, 1), jnp.float32),
            jax.ShapeDtypeStruct((1, C), jnp.float32),
        ],
    )(xp, W1a, W1b, deg2d, t2, Wt, bt2)


def _ln(z, g, b):
    mu = jnp.mean(z, axis=-1, keepdims=True)
    var = jnp.mean((z - mu) ** 2, axis=-1, keepdims=True)
    return (z - mu) * lax.rsqrt(var + 1e-5) * g + b


def _tcb_body(msg_ref, b1_ref, g1_ref, be1_ref, temb_ref, wa_ref, wb_ref,
              dis_ref, y2_ref):
    z = jax.nn.relu(msg_ref[...] + b1_ref[...])
    h = _ln(z, g1_ref[...], be1_ref[...]) + temb_ref[...]
    d = dis_ref[...]
    a = jnp.dot(h, wa_ref[...], preferred_element_type=jnp.float32) * d
    b = jnp.dot(h, wb_ref[...], preferred_element_type=jnp.float32) * d
    y2_ref[...] = _pack_rows(a, b)


@jax.jit
def _tc_b(msg1, b1r, g1r, be1r, temb, W2a, W2b, dis1):
    return pl.pallas_call(
        _tcb_body,
        grid=(NP // 1024,),
        in_specs=[
            pl.BlockSpec((1024, C), lambda i: (i, 0)),
            pl.BlockSpec((1, C), lambda i: (0, 0)),
            pl.BlockSpec((1, C), lambda i: (0, 0)),
            pl.BlockSpec((1, C), lambda i: (0, 0)),
            pl.BlockSpec((1, C), lambda i: (0, 0)),
            pl.BlockSpec((C, CW), lambda i: (0, 0)),
            pl.BlockSpec((C, CW), lambda i: (0, 0)),
            pl.BlockSpec((1024, 1), lambda i: (i, 0)),
        ],
        out_specs=pl.BlockSpec((1024, CW), lambda i: (i, 0)),
        out_shape=jax.ShapeDtypeStruct((NP, CW), jnp.int32),
    )(msg1, b1r, g1r, be1r, temb, W2a, W2b, dis1)


def _tcc_body(msg_ref, b2_ref, g2_ref, be2_ref, out_ref):
    z = jax.nn.relu(msg_ref[...] + b2_ref[...])
    out_ref[...] = _ln(z, g2_ref[...], be2_ref[...])


@jax.jit
def _tc_c(msg2, b2r, g2r, be2r):
    return pl.pallas_call(
        _tcc_body,
        grid=(NP // 1024,),
        in_specs=[
            pl.BlockSpec((1024, C), lambda i: (i, 0)),
            pl.BlockSpec((1, C), lambda i: (0, 0)),
            pl.BlockSpec((1, C), lambda i: (0, 0)),
            pl.BlockSpec((1, C), lambda i: (0, 0)),
        ],
        out_specs=pl.BlockSpec((1024, C), lambda i: (i, 0)),
        out_shape=jax.ShapeDtypeStruct((NP, C), jnp.float32),
    )(msg2, b2r, g2r, be2r)


def kernel(x, edge_index, edge_weight, t, W1, b1, g1, be1, W2, b2, g2, be2,
           Wt, bt):
    xp = jnp.pad(x, ((0, NP - N), (0, 0)))
    ms = jnp.arange(CW, dtype=jnp.int32)
    sig_a = 32 * (ms // 16) + (ms % 16)
    sig_b = sig_a + 16
    rows_s, cols_s, ews_s, counts, deg = _sc_partition(
        edge_index[0], edge_index[1], edge_weight)
    y1, dis2d, temb = _tc_a(xp, W1[:, sig_a], W1[:, sig_b],
                            deg.reshape(NP, 1), t.reshape(1, C), Wt,
                            bt.reshape(1, C))
    dis = dis2d.reshape(NP)
    msg1 = _sc_conv(y1, rows_s, cols_s, ews_s, counts, dis)
    y2 = _tc_b(msg1, b1.reshape(1, C), g1.reshape(1, C), be1.reshape(1, C),
               temb, W2[:, sig_a], W2[:, sig_b], dis2d)
    msg2 = _sc_conv(y2, rows_s, cols_s, ews_s, counts, dis)
    out = _tc_c(msg2, b2.reshape(1, C), g2.reshape(1, C), be2.reshape(1, C))
    return out[:N]
